# Initial kernel scaffold; baseline (speedup 1.0000x reference)
import jax
import jax.numpy as jnp
from jax.experimental import pallas as pl


def kernel(x, edge_index, edge_attr, W_src, b_src, W_dst, b_dst, W_edge, b_edge, W_msg, b_msg, W_res, b_res, W_eo, b_eo, gamma_n, beta_n, gamma_e, beta_e):
    raise NotImplementedError("write your pallas kernel here")


# speedup vs baseline: 1.0110x; 1.0110x over previous
import jax
import jax.numpy as jnp
from jax import lax
from jax.experimental import pallas as pl
from jax.experimental.pallas import tpu as pltpu
from jax.experimental.pallas import tpu_sc as plsc

N = 10000
E = 320000
D = 128
H = D // 2

NUM_TILES = 16
EPT = E // NUM_TILES
C = 80
NCHUNK = EPT // C
NPAD = 10240
NPT = NPAD // NUM_TILES

_F32 = jnp.float32


def _node_fwd_body(x_ref, ws_ref, wd_ref, wm_ref, wr_ref,
                   bs_ref, bd_ref, bm_ref, br_ref,
                   tsm_lo, tsm_hi, xd_ref, xr_ref):
    xb = x_ref[...]

    def lin(w_ref, b_ref):
        return jnp.dot(xb, w_ref[...], preferred_element_type=_F32) + b_ref[...]

    xs = lin(ws_ref, bs_ref)
    xm = lin(wm_ref, bm_ref)
    tsm_lo[...] = jnp.concatenate([xs[:, :H], xm[:, :H]], axis=1)
    tsm_hi[...] = jnp.concatenate([xs[:, H:], xm[:, H:]], axis=1)
    xd_ref[...] = lin(wd_ref, bd_ref)
    xr_ref[...] = lin(wr_ref, br_ref)


def _node_fwd(x, W_src, W_dst, W_msg, W_res, bs, bd, bm, br):
    bn = 1000
    grid = (N // bn,)
    w_spec = pl.BlockSpec((D, D), lambda i: (0, 0))
    b_spec = pl.BlockSpec((1, D), lambda i: (0, 0))
    full = pl.BlockSpec((bn, D), lambda i: (i, 0))
    return pl.pallas_call(
        _node_fwd_body,
        grid=grid,
        in_specs=[full, w_spec, w_spec, w_spec, w_spec,
                  b_spec, b_spec, b_spec, b_spec],
        out_specs=[full, full, full, full],
        out_shape=[jax.ShapeDtypeStruct((N, D), _F32)] * 4,
    )(x, W_src, W_dst, W_msg, W_res, bs, bd, bm, br)


def _edge_fwd_body(a_ref, w_ref, b_ref, lo_ref, hi_ref):
    ea = (jnp.dot(a_ref[...], w_ref[...], preferred_element_type=_F32)
          + b_ref[...])
    lo_ref[...] = ea[:, :H]
    hi_ref[...] = ea[:, H:]


def _edge_fwd(edge_attr, W_edge, be):
    bn = 4000
    grid = (E // bn,)
    return pl.pallas_call(
        _edge_fwd_body,
        grid=grid,
        in_specs=[pl.BlockSpec((bn, D), lambda i: (i, 0)),
                  pl.BlockSpec((D, D), lambda i: (0, 0)),
                  pl.BlockSpec((1, D), lambda i: (0, 0))],
        out_specs=[pl.BlockSpec((bn, H), lambda i: (i, 0)),
                   pl.BlockSpec((bn, H), lambda i: (i, 0))],
        out_shape=[jax.ShapeDtypeStruct((E, H), _F32)] * 2,
    )(edge_attr, W_edge, be)


ZB = 64


def _sc_gate_body(src, dst, tsm_lo, tsm_hi, xd_t, ea_lo, ea_hi,
                  gate_lo, gate_hi, gs_lo, gs_hi, ms_lo, ms_hi,
                  src_idx, dst_idx, sm_g, xd_g, ea_b, gate_b, msg_b,
                  zero_b, acc_g, acc_m, sem):
    cid = lax.axis_index("c")
    sid = lax.axis_index("s")

    def run_half(tsm_t, ea_t, gate_out, gs_out, ms_out, off):
        def zfill(r, _):
            for k in range(H // 16):
                zero_b[r, pl.ds(k * 16, 16)] = jnp.zeros((16,), _F32)
            return 0
        lax.fori_loop(0, ZB, zfill, 0)
        row0 = sid * NPT

        def zcopy(j, _):
            pltpu.sync_copy(zero_b, acc_g.at[pl.ds(row0 + j * ZB, ZB)])
            pltpu.sync_copy(zero_b, acc_m.at[pl.ds(row0 + j * ZB, ZB)])
            return 0
        lax.fori_loop(0, NPT // ZB, zcopy, 0)
        plsc.subcore_barrier()

        def chunk(i, _):
            base = sid * EPT + i * C
            pltpu.sync_copy(src.at[pl.ds(base, C)], src_idx)
            pltpu.sync_copy(dst.at[pl.ds(base, C)], dst_idx)
            cp1 = pltpu.async_copy(tsm_t.at[src_idx], sm_g, sem)
            cp2 = pltpu.async_copy(xd_t.at[dst_idx], xd_g, sem)
            pltpu.sync_copy(ea_t.at[pl.ds(base, C)], ea_b)
            cp1.wait()
            cp2.wait()

            def comp(r, _):
                for k in range(H // 16):
                    sl = pl.ds(k * 16, 16)
                    t = (sm_g[r, sl] + xd_g[r, pl.ds(off + k * 16, 16)]
                         + ea_b[r, sl])
                    g = 1.0 / (1.0 + jnp.exp(-t))
                    gate_b[r, sl] = g
                    msg_b[r, sl] = sm_g[r, pl.ds(H + k * 16, 16)] * g
                return 0
            lax.fori_loop(0, C, comp, 0)

            pltpu.sync_copy(gate_b, gate_out.at[pl.ds(base, C)])
            pltpu.sync_copy(gate_b, acc_g.at[dst_idx], add=True)
            pltpu.sync_copy(msg_b, acc_m.at[dst_idx], add=True)
            return 0
        lax.fori_loop(0, NCHUNK, chunk, 0)

        plsc.subcore_barrier()
        pltpu.sync_copy(acc_g.at[pl.ds(row0, NPT)], gs_out.at[pl.ds(row0, NPT)])
        pltpu.sync_copy(acc_m.at[pl.ds(row0, NPT)], ms_out.at[pl.ds(row0, NPT)])

    @pl.when(cid == 0)
    def _():
        run_half(tsm_lo, ea_lo, gate_lo, gs_lo, ms_lo, 0)

    @pl.when(cid == 1)
    def _():
        run_half(tsm_hi, ea_hi, gate_hi, gs_hi, ms_hi, H)


def _sc_gate(src, dst, tsm_lo, tsm_hi, xd, ea_lo, ea_hi):
    mesh = plsc.VectorSubcoreMesh(core_axis_name="c", subcore_axis_name="s")
    f = pl.kernel(
        _sc_gate_body,
        out_type=[jax.ShapeDtypeStruct((E, H), _F32)] * 2
        + [jax.ShapeDtypeStruct((NPAD, H), _F32)] * 4,
        mesh=mesh,
        scratch_types=[
            pltpu.VMEM((C,), jnp.int32),
            pltpu.VMEM((C,), jnp.int32),
            pltpu.VMEM((C, D), _F32),
            pltpu.VMEM((C, D), _F32),
            pltpu.VMEM((C, H), _F32),
            pltpu.VMEM((C, H), _F32),
            pltpu.VMEM((C, H), _F32),
            pltpu.VMEM((ZB, H), _F32),
            pltpu.VMEM_SHARED((NPAD, H), _F32),
            pltpu.VMEM_SHARED((NPAD, H), _F32),
            pltpu.SemaphoreType.DMA,
        ],
        compiler_params=pltpu.CompilerParams(use_tc_tiling_on_sc=False),
    )
    return f(src, dst, tsm_lo, tsm_hi, xd, ea_lo, ea_hi)


def _edge_stats_body(glo_ref, ghi_ref, w_ref, b_ref, stats_ref, acc_ref):
    i = pl.program_id(0)

    @pl.when(i == 0)
    def _():
        acc_ref[...] = jnp.zeros_like(acc_ref)

    w = w_ref[...]
    h = (jnp.dot(glo_ref[...], w[:H, :], preferred_element_type=_F32)
         + jnp.dot(ghi_ref[...], w[H:, :], preferred_element_type=_F32)
         + b_ref[...])
    acc_ref[0:1, :] += jnp.sum(h, axis=0, keepdims=True)
    acc_ref[1:2, :] += jnp.sum(h * h, axis=0, keepdims=True)

    @pl.when(i == pl.num_programs(0) - 1)
    def _():
        stats_ref[...] = acc_ref[...]


def _edge_stats(gate_lo, gate_hi, W_eo, beo):
    bn = 4000
    grid = (E // bn,)
    return pl.pallas_call(
        _edge_stats_body,
        grid=grid,
        in_specs=[pl.BlockSpec((bn, H), lambda i: (i, 0)),
                  pl.BlockSpec((bn, H), lambda i: (i, 0)),
                  pl.BlockSpec((D, D), lambda i: (0, 0)),
                  pl.BlockSpec((1, D), lambda i: (0, 0))],
        out_specs=pl.BlockSpec((8, D), lambda i: (0, 0)),
        out_shape=jax.ShapeDtypeStruct((8, D), _F32),
        scratch_shapes=[pltpu.VMEM((8, D), _F32)],
    )(gate_lo, gate_hi, W_eo, beo)


def _edge_out_body(glo_ref, ghi_ref, w_ref, b_ref, stats_ref, gam_ref,
                   bt_ref, out_ref):
    w = w_ref[...]
    h = (jnp.dot(glo_ref[...], w[:H, :], preferred_element_type=_F32)
         + jnp.dot(ghi_ref[...], w[H:, :], preferred_element_type=_F32)
         + b_ref[...])
    mu = stats_ref[0:1, :] / E
    var = stats_ref[1:2, :] / E - mu * mu
    scale = gam_ref[...] * lax.rsqrt(var + 1e-5)
    out_ref[...] = jnp.maximum((h - mu) * scale + bt_ref[...], 0.0)


def _edge_out(gate_lo, gate_hi, W_eo, beo, stats, ge, bte):
    bn = 4000
    grid = (E // bn,)
    return pl.pallas_call(
        _edge_out_body,
        grid=grid,
        in_specs=[pl.BlockSpec((bn, H), lambda i: (i, 0)),
                  pl.BlockSpec((bn, H), lambda i: (i, 0)),
                  pl.BlockSpec((D, D), lambda i: (0, 0)),
                  pl.BlockSpec((1, D), lambda i: (0, 0)),
                  pl.BlockSpec((8, D), lambda i: (0, 0)),
                  pl.BlockSpec((1, D), lambda i: (0, 0)),
                  pl.BlockSpec((1, D), lambda i: (0, 0))],
        out_specs=pl.BlockSpec((bn, D), lambda i: (i, 0)),
        out_shape=jax.ShapeDtypeStruct((E, D), _F32),
    )(gate_lo, gate_hi, W_eo, beo, stats, ge, bte)


def _node_out_body(xr_ref, gslo_ref, gshi_ref, mslo_ref, mshi_ref,
                   gam_ref, bt_ref, out_ref):
    agg_lo = mslo_ref[...] / (gslo_ref[...] + 1e-6)
    agg_hi = mshi_ref[...] / (gshi_ref[...] + 1e-6)
    t = xr_ref[...] + jnp.concatenate([agg_lo, agg_hi], axis=1)
    mu = jnp.mean(t, axis=0, keepdims=True)
    var = jnp.mean((t - mu) ** 2, axis=0, keepdims=True)
    norm = gam_ref[...] * (t - mu) * lax.rsqrt(var + 1e-5) + bt_ref[...]
    out_ref[...] = jnp.maximum(norm, 0.0)


def _node_out(xr, gs_lo, gs_hi, ms_lo, ms_hi, gn, btn):
    half = pl.BlockSpec((N, H), lambda i: (0, 0))
    return pl.pallas_call(
        _node_out_body,
        grid=(1,),
        in_specs=[pl.BlockSpec((N, D), lambda i: (0, 0)),
                  half, half, half, half,
                  pl.BlockSpec((1, D), lambda i: (0, 0)),
                  pl.BlockSpec((1, D), lambda i: (0, 0))],
        out_specs=pl.BlockSpec((N, D), lambda i: (0, 0)),
        out_shape=jax.ShapeDtypeStruct((N, D), _F32),
    )(xr, gs_lo, gs_hi, ms_lo, ms_hi, gn, btn)


@jax.jit
def kernel(x, edge_index, edge_attr, W_src, b_src, W_dst, b_dst, W_edge,
           b_edge, W_msg, b_msg, W_res, b_res, W_eo, b_eo, gamma_n, beta_n,
           gamma_e, beta_e):
    r = lambda b: b.reshape(1, D)
    tsm_lo, tsm_hi, xd, xr = _node_fwd(
        x, W_src, W_dst, W_msg, W_res, r(b_src), r(b_dst), r(b_msg), r(b_res))
    ea_lo, ea_hi = _edge_fwd(edge_attr, W_edge, r(b_edge))
    gate_lo, gate_hi, gs_lo, gs_hi, ms_lo, ms_hi = _sc_gate(
        edge_index[0], edge_index[1], tsm_lo, tsm_hi, xd, ea_lo, ea_hi)
    stats = _edge_stats(gate_lo, gate_hi, W_eo, r(b_eo))
    edge_new = _edge_out(gate_lo, gate_hi, W_eo, r(b_eo), stats,
                         r(gamma_e), r(beta_e))
    x_out = _node_out(xr, gs_lo, gs_hi, ms_lo, ms_hi, r(gamma_n), r(beta_n))
    return (x_out, edge_new)


# speedup vs baseline: 1.0819x; 1.0702x over previous
import jax
import jax.numpy as jnp
from jax import lax
from jax.experimental import pallas as pl
from jax.experimental.pallas import tpu as pltpu
from jax.experimental.pallas import tpu_sc as plsc

N = 10000
E = 320000
D = 128
H = D // 2

NUM_TILES = 16
EPT = E // NUM_TILES
C = 40
NCHUNK = EPT // C
NPAD = 10240
NPT = NPAD // NUM_TILES

_F32 = jnp.float32


def _node_fwd_body(x_ref, ws_ref, wd_ref, wm_ref, wr_ref,
                   bs_ref, bd_ref, bm_ref, br_ref,
                   tsm_lo, tsm_hi, xd_lo, xd_hi, xr_ref):
    xb = x_ref[...]

    def lin(w_ref, b_ref):
        return jnp.dot(xb, w_ref[...], preferred_element_type=_F32) + b_ref[...]

    xs = lin(ws_ref, bs_ref)
    xm = lin(wm_ref, bm_ref)
    tsm_lo[...] = jnp.concatenate([xs[:, :H], xm[:, :H]], axis=1)
    tsm_hi[...] = jnp.concatenate([xs[:, H:], xm[:, H:]], axis=1)
    xd = lin(wd_ref, bd_ref)
    xd_lo[...] = xd[:, :H]
    xd_hi[...] = xd[:, H:]
    xr_ref[...] = lin(wr_ref, br_ref)


def _node_fwd(x, W_src, W_dst, W_msg, W_res, bs, bd, bm, br):
    bn = 1000
    grid = (N // bn,)
    w_spec = pl.BlockSpec((D, D), lambda i: (0, 0))
    b_spec = pl.BlockSpec((1, D), lambda i: (0, 0))
    full = pl.BlockSpec((bn, D), lambda i: (i, 0))
    half = pl.BlockSpec((bn, H), lambda i: (i, 0))
    return pl.pallas_call(
        _node_fwd_body,
        grid=grid,
        in_specs=[full, w_spec, w_spec, w_spec, w_spec,
                  b_spec, b_spec, b_spec, b_spec],
        out_specs=[full, full, half, half, full],
        out_shape=[jax.ShapeDtypeStruct((N, D), _F32)] * 2
        + [jax.ShapeDtypeStruct((N, H), _F32)] * 2
        + [jax.ShapeDtypeStruct((N, D), _F32)],
    )(x, W_src, W_dst, W_msg, W_res, bs, bd, bm, br)


def _edge_fwd_body(a_ref, w_ref, b_ref, lo_ref, hi_ref):
    ea = (jnp.dot(a_ref[...], w_ref[...], preferred_element_type=_F32)
          + b_ref[...])
    lo_ref[...] = ea[:, :H]
    hi_ref[...] = ea[:, H:]


def _edge_fwd(edge_attr, W_edge, be):
    bn = 4000
    grid = (E // bn,)
    return pl.pallas_call(
        _edge_fwd_body,
        grid=grid,
        in_specs=[pl.BlockSpec((bn, D), lambda i: (i, 0)),
                  pl.BlockSpec((D, D), lambda i: (0, 0)),
                  pl.BlockSpec((1, D), lambda i: (0, 0))],
        out_specs=[pl.BlockSpec((bn, H), lambda i: (i, 0)),
                   pl.BlockSpec((bn, H), lambda i: (i, 0))],
        out_shape=[jax.ShapeDtypeStruct((E, H), _F32)] * 2,
    )(edge_attr, W_edge, be)


def _sc_gate_body(src, dst, tsm_lo, tsm_hi, xd_lo, xd_hi, ea_lo, ea_hi,
                  gate_lo, gate_hi, gs_lo, gs_hi, ms_lo, ms_hi,
                  src_idx, dst_idx, sm_g, xd_g, ea_b, gate_b, msg_b,
                  acc_g, acc_m, sem0, sem1, gw0, gw1):
    cid = lax.axis_index("c")
    sid = lax.axis_index("s")

    def run_half(tsm_t, xd_t, ea_t, gate_out, gs_out, ms_out):
        sems = (sem0, sem1)
        gws = (gw0, gw1)
        tile_base = sid * EPT

        def zfill(r, _):
            for k in range(H // 16):
                msg_b[r, pl.ds(k * 16, 16)] = jnp.zeros((16,), _F32)
            return 0
        lax.fori_loop(0, C, zfill, 0)
        row0 = sid * NPT

        def zcopy(j, _):
            pltpu.sync_copy(msg_b, acc_g.at[pl.ds(row0 + j * C, C)])
            pltpu.sync_copy(msg_b, acc_m.at[pl.ds(row0 + j * C, C)])
            return 0
        lax.fori_loop(0, NPT // C, zcopy, 0)
        plsc.subcore_barrier()

        def issue(b, base):
            pltpu.sync_copy(src.at[pl.ds(base, C)], src_idx.at[b])
            pltpu.sync_copy(dst.at[pl.ds(base, C)], dst_idx.at[b])
            pltpu.async_copy(tsm_t.at[src_idx.at[b]], sm_g.at[b], sems[b])
            pltpu.async_copy(xd_t.at[dst_idx.at[b]], xd_g.at[b], sems[b])
            pltpu.async_copy(ea_t.at[pl.ds(base, C)], ea_b.at[b], sems[b])

        def wait_in(b):
            pltpu.make_async_copy(
                tsm_t.at[src_idx.at[b]], sm_g.at[b], sems[b]).wait()
            pltpu.make_async_copy(
                xd_t.at[dst_idx.at[b]], xd_g.at[b], sems[b]).wait()
            pltpu.make_async_copy(
                ea_t.at[pl.ds(0, C)], ea_b.at[b], sems[b]).wait()

        def do_chunk(b, i, base):
            wait_in(b)

            @pl.when(i >= 2)
            def _():
                pltpu.make_async_copy(
                    gate_b.at[b], gate_out.at[pl.ds(0, C)], gws[b]).wait()

            def comp(r, _):
                for k in range(H // 16):
                    sl = pl.ds(k * 16, 16)
                    t = sm_g[b, r, sl] + xd_g[b, r, sl] + ea_b[b, r, sl]
                    g = 1.0 / (1.0 + jnp.exp(-t))
                    gate_b[b, r, sl] = g
                    msg_b[r, sl] = sm_g[b, r, pl.ds(H + k * 16, 16)] * g
                return 0
            lax.fori_loop(0, C, comp, 0)

            pltpu.sync_copy(gate_b.at[b], acc_g.at[dst_idx.at[b]], add=True)
            pltpu.sync_copy(msg_b, acc_m.at[dst_idx.at[b]], add=True)
            pltpu.async_copy(gate_b.at[b], gate_out.at[pl.ds(base, C)], gws[b])

        issue(0, tile_base)

        def step(g, _):
            i0 = 2 * g
            issue(1, tile_base + lax.rem(i0 + 1, NCHUNK) * C)
            do_chunk(0, i0, tile_base + i0 * C)
            issue(0, tile_base + lax.rem(i0 + 2, NCHUNK) * C)
            do_chunk(1, i0 + 1, tile_base + (i0 + 1) * C)
            return 0
        lax.fori_loop(0, NCHUNK // 2, step, 0)

        wait_in(0)
        pltpu.make_async_copy(
            gate_b.at[0], gate_out.at[pl.ds(0, C)], gws[0]).wait()
        pltpu.make_async_copy(
            gate_b.at[1], gate_out.at[pl.ds(0, C)], gws[1]).wait()

        plsc.subcore_barrier()
        pltpu.sync_copy(acc_g.at[pl.ds(row0, NPT)], gs_out.at[pl.ds(row0, NPT)])
        pltpu.sync_copy(acc_m.at[pl.ds(row0, NPT)], ms_out.at[pl.ds(row0, NPT)])

    @pl.when(cid == 0)
    def _():
        run_half(tsm_lo, xd_lo, ea_lo, gate_lo, gs_lo, ms_lo)

    @pl.when(cid == 1)
    def _():
        run_half(tsm_hi, xd_hi, ea_hi, gate_hi, gs_hi, ms_hi)


def _sc_gate(src, dst, tsm_lo, tsm_hi, xd_lo, xd_hi, ea_lo, ea_hi):
    mesh = plsc.VectorSubcoreMesh(core_axis_name="c", subcore_axis_name="s")
    f = pl.kernel(
        _sc_gate_body,
        out_type=[jax.ShapeDtypeStruct((E, H), _F32)] * 2
        + [jax.ShapeDtypeStruct((NPAD, H), _F32)] * 4,
        mesh=mesh,
        scratch_types=[
            pltpu.VMEM((2, C), jnp.int32),
            pltpu.VMEM((2, C), jnp.int32),
            pltpu.VMEM((2, C, D), _F32),
            pltpu.VMEM((2, C, H), _F32),
            pltpu.VMEM((2, C, H), _F32),
            pltpu.VMEM((2, C, H), _F32),
            pltpu.VMEM((C, H), _F32),
            pltpu.VMEM_SHARED((NPAD, H), _F32),
            pltpu.VMEM_SHARED((NPAD, H), _F32),
            pltpu.SemaphoreType.DMA,
            pltpu.SemaphoreType.DMA,
            pltpu.SemaphoreType.DMA,
            pltpu.SemaphoreType.DMA,
        ],
        compiler_params=pltpu.CompilerParams(use_tc_tiling_on_sc=False),
    )
    return f(src, dst, tsm_lo, tsm_hi, xd_lo, xd_hi, ea_lo, ea_hi)


def _edge_stats_body(glo_ref, ghi_ref, w_ref, b_ref, stats_ref, acc_ref):
    i = pl.program_id(0)

    @pl.when(i == 0)
    def _():
        acc_ref[...] = jnp.zeros_like(acc_ref)

    w = w_ref[...]
    h = (jnp.dot(glo_ref[...], w[:H, :], preferred_element_type=_F32)
         + jnp.dot(ghi_ref[...], w[H:, :], preferred_element_type=_F32)
         + b_ref[...])
    acc_ref[0:1, :] += jnp.sum(h, axis=0, keepdims=True)
    acc_ref[1:2, :] += jnp.sum(h * h, axis=0, keepdims=True)

    @pl.when(i == pl.num_programs(0) - 1)
    def _():
        stats_ref[...] = acc_ref[...]


def _edge_stats(gate_lo, gate_hi, W_eo, beo):
    bn = 4000
    grid = (E // bn,)
    return pl.pallas_call(
        _edge_stats_body,
        grid=grid,
        in_specs=[pl.BlockSpec((bn, H), lambda i: (i, 0)),
                  pl.BlockSpec((bn, H), lambda i: (i, 0)),
                  pl.BlockSpec((D, D), lambda i: (0, 0)),
                  pl.BlockSpec((1, D), lambda i: (0, 0))],
        out_specs=pl.BlockSpec((8, D), lambda i: (0, 0)),
        out_shape=jax.ShapeDtypeStruct((8, D), _F32),
        scratch_shapes=[pltpu.VMEM((8, D), _F32)],
    )(gate_lo, gate_hi, W_eo, beo)


def _edge_out_body(glo_ref, ghi_ref, w_ref, b_ref, stats_ref, gam_ref,
                   bt_ref, out_ref):
    w = w_ref[...]
    h = (jnp.dot(glo_ref[...], w[:H, :], preferred_element_type=_F32)
         + jnp.dot(ghi_ref[...], w[H:, :], preferred_element_type=_F32)
         + b_ref[...])
    mu = stats_ref[0:1, :] / E
    var = stats_ref[1:2, :] / E - mu * mu
    scale = gam_ref[...] * lax.rsqrt(var + 1e-5)
    out_ref[...] = jnp.maximum((h - mu) * scale + bt_ref[...], 0.0)


def _edge_out(gate_lo, gate_hi, W_eo, beo, stats, ge, bte):
    bn = 4000
    grid = (E // bn,)
    return pl.pallas_call(
        _edge_out_body,
        grid=grid,
        in_specs=[pl.BlockSpec((bn, H), lambda i: (i, 0)),
                  pl.BlockSpec((bn, H), lambda i: (i, 0)),
                  pl.BlockSpec((D, D), lambda i: (0, 0)),
                  pl.BlockSpec((1, D), lambda i: (0, 0)),
                  pl.BlockSpec((8, D), lambda i: (0, 0)),
                  pl.BlockSpec((1, D), lambda i: (0, 0)),
                  pl.BlockSpec((1, D), lambda i: (0, 0))],
        out_specs=pl.BlockSpec((bn, D), lambda i: (i, 0)),
        out_shape=jax.ShapeDtypeStruct((E, D), _F32),
    )(gate_lo, gate_hi, W_eo, beo, stats, ge, bte)


def _node_out_body(xr_ref, gslo_ref, gshi_ref, mslo_ref, mshi_ref,
                   gam_ref, bt_ref, out_ref):
    agg_lo = mslo_ref[...] / (gslo_ref[...] + 1e-6)
    agg_hi = mshi_ref[...] / (gshi_ref[...] + 1e-6)
    t = xr_ref[...] + jnp.concatenate([agg_lo, agg_hi], axis=1)
    mu = jnp.mean(t, axis=0, keepdims=True)
    var = jnp.mean((t - mu) ** 2, axis=0, keepdims=True)
    norm = gam_ref[...] * (t - mu) * lax.rsqrt(var + 1e-5) + bt_ref[...]
    out_ref[...] = jnp.maximum(norm, 0.0)


def _node_out(xr, gs_lo, gs_hi, ms_lo, ms_hi, gn, btn):
    half = pl.BlockSpec((N, H), lambda i: (0, 0))
    return pl.pallas_call(
        _node_out_body,
        grid=(1,),
        in_specs=[pl.BlockSpec((N, D), lambda i: (0, 0)),
                  half, half, half, half,
                  pl.BlockSpec((1, D), lambda i: (0, 0)),
                  pl.BlockSpec((1, D), lambda i: (0, 0))],
        out_specs=pl.BlockSpec((N, D), lambda i: (0, 0)),
        out_shape=jax.ShapeDtypeStruct((N, D), _F32),
    )(xr, gs_lo, gs_hi, ms_lo, ms_hi, gn, btn)


@jax.jit
def kernel(x, edge_index, edge_attr, W_src, b_src, W_dst, b_dst, W_edge,
           b_edge, W_msg, b_msg, W_res, b_res, W_eo, b_eo, gamma_n, beta_n,
           gamma_e, beta_e):
    r = lambda b: b.reshape(1, D)
    tsm_lo, tsm_hi, xd_lo, xd_hi, xr = _node_fwd(
        x, W_src, W_dst, W_msg, W_res, r(b_src), r(b_dst), r(b_msg), r(b_res))
    ea_lo, ea_hi = _edge_fwd(edge_attr, W_edge, r(b_edge))
    gate_lo, gate_hi, gs_lo, gs_hi, ms_lo, ms_hi = _sc_gate(
        edge_index[0], edge_index[1], tsm_lo, tsm_hi, xd_lo, xd_hi,
        ea_lo, ea_hi)
    stats = _edge_stats(gate_lo, gate_hi, W_eo, r(b_eo))
    edge_new = _edge_out(gate_lo, gate_hi, W_eo, r(b_eo), stats,
                         r(gamma_e), r(beta_e))
    x_out = _node_out(xr, gs_lo, gs_hi, ms_lo, ms_hi, r(gamma_n), r(beta_n))
    return (x_out, edge_new)


# speedup vs baseline: 1.1436x; 1.0570x over previous
import jax
import jax.numpy as jnp
from jax import lax
from jax.experimental import pallas as pl
from jax.experimental.pallas import tpu as pltpu
from jax.experimental.pallas import tpu_sc as plsc

N = 10000
E = 320000
D = 128
H = D // 2

NUM_TILES = 16
EPT = E // NUM_TILES
C = 40
NCHUNK = EPT // C
NPAD = 10240
NPT = NPAD // NUM_TILES

_F32 = jnp.float32


def _node_fwd_body(x_ref, ws_ref, wd_ref, wm_ref, wr_ref,
                   bs_ref, bd_ref, bm_ref, br_ref,
                   tsm_lo, tsm_hi, xd_lo, xd_hi, xr_ref):
    xb = x_ref[...]

    def lin(w_ref, b_ref):
        return jnp.dot(xb, w_ref[...], preferred_element_type=_F32) + b_ref[...]

    xs = lin(ws_ref, bs_ref)
    xm = lin(wm_ref, bm_ref)
    tsm_lo[...] = jnp.concatenate([xs[:, :H], xm[:, :H]], axis=1)
    tsm_hi[...] = jnp.concatenate([xs[:, H:], xm[:, H:]], axis=1)
    xd = lin(wd_ref, bd_ref)
    xd_lo[...] = xd[:, :H]
    xd_hi[...] = xd[:, H:]
    xr_ref[...] = lin(wr_ref, br_ref)


def _node_fwd(x, W_src, W_dst, W_msg, W_res, bs, bd, bm, br):
    bn = 1000
    grid = (N // bn,)
    w_spec = pl.BlockSpec((D, D), lambda i: (0, 0))
    b_spec = pl.BlockSpec((1, D), lambda i: (0, 0))
    full = pl.BlockSpec((bn, D), lambda i: (i, 0))
    half = pl.BlockSpec((bn, H), lambda i: (i, 0))
    return pl.pallas_call(
        _node_fwd_body,
        grid=grid,
        in_specs=[full, w_spec, w_spec, w_spec, w_spec,
                  b_spec, b_spec, b_spec, b_spec],
        out_specs=[full, full, half, half, full],
        out_shape=[jax.ShapeDtypeStruct((N, D), _F32)] * 2
        + [jax.ShapeDtypeStruct((N, H), _F32)] * 2
        + [jax.ShapeDtypeStruct((N, D), _F32)],
    )(x, W_src, W_dst, W_msg, W_res, bs, bd, bm, br)


def _edge_fwd_body(a_ref, w_ref, b_ref, lo_ref, hi_ref):
    ea = (jnp.dot(a_ref[...], w_ref[...], preferred_element_type=_F32)
          + b_ref[...])
    lo_ref[...] = ea[:, :H]
    hi_ref[...] = ea[:, H:]


def _edge_fwd(edge_attr, W_edge, be):
    bn = 4000
    grid = (E // bn,)
    return pl.pallas_call(
        _edge_fwd_body,
        grid=grid,
        in_specs=[pl.BlockSpec((bn, D), lambda i: (i, 0)),
                  pl.BlockSpec((D, D), lambda i: (0, 0)),
                  pl.BlockSpec((1, D), lambda i: (0, 0))],
        out_specs=[pl.BlockSpec((bn, H), lambda i: (i, 0)),
                   pl.BlockSpec((bn, H), lambda i: (i, 0))],
        out_shape=[jax.ShapeDtypeStruct((E, H), _F32)] * 2,
    )(edge_attr, W_edge, be)


def _sc_gate_body(src, dst, tsm_lo, tsm_hi, xd_lo, xd_hi, ea_lo, ea_hi,
                  gate_lo, gate_hi, gs_lo, gs_hi, ms_lo, ms_hi,
                  src_idx, dst_idx, sm_g, xd_g, ea_b, gate_b, msg_b,
                  acc_g, acc_m, sem0, sem1, gw0, gw1, isem0, isem1):
    cid = lax.axis_index("c")
    sid = lax.axis_index("s")

    def run_half(tsm_t, xd_t, ea_t, gate_out, gs_out, ms_out):
        sems = (sem0, sem1)
        gws = (gw0, gw1)
        isems = (isem0, isem1)
        tile_base = sid * EPT

        def zfill(r, _):
            for k in range(H // 16):
                msg_b[r, pl.ds(k * 16, 16)] = jnp.zeros((16,), _F32)
            return 0
        lax.fori_loop(0, C, zfill, 0)
        row0 = sid * NPT

        def zcopy(j, _):
            pltpu.sync_copy(msg_b, acc_g.at[pl.ds(row0 + j * C, C)])
            pltpu.sync_copy(msg_b, acc_m.at[pl.ds(row0 + j * C, C)])
            return 0
        lax.fori_loop(0, NPT // C, zcopy, 0)
        plsc.subcore_barrier()

        def issue_idx(b, base):
            pltpu.async_copy(src.at[pl.ds(base, C)], src_idx.at[b], isems[b])
            pltpu.async_copy(dst.at[pl.ds(base, C)], dst_idx.at[b], isems[b])

        def issue_gather(b, base):
            pltpu.make_async_copy(
                src.at[pl.ds(0, C)], src_idx.at[b], isems[b]).wait()
            pltpu.make_async_copy(
                dst.at[pl.ds(0, C)], dst_idx.at[b], isems[b]).wait()
            pltpu.async_copy(tsm_t.at[src_idx.at[b]], sm_g.at[b], sems[b])
            pltpu.async_copy(xd_t.at[dst_idx.at[b]], xd_g.at[b], sems[b])
            pltpu.async_copy(ea_t.at[pl.ds(base, C)], ea_b.at[b], sems[b])

        def wait_in(b):
            pltpu.make_async_copy(
                tsm_t.at[src_idx.at[b]], sm_g.at[b], sems[b]).wait()
            pltpu.make_async_copy(
                xd_t.at[dst_idx.at[b]], xd_g.at[b], sems[b]).wait()
            pltpu.make_async_copy(
                ea_t.at[pl.ds(0, C)], ea_b.at[b], sems[b]).wait()

        def do_chunk(b, i, base):
            wait_in(b)

            @pl.when(i >= 2)
            def _():
                pltpu.make_async_copy(
                    gate_b.at[b], gate_out.at[pl.ds(0, C)], gws[b]).wait()

            def comp(rr, _):
                for r in (2 * rr, 2 * rr + 1):
                    for k in range(H // 16):
                        sl = pl.ds(k * 16, 16)
                        t = sm_g[b, r, sl] + xd_g[b, r, sl] + ea_b[b, r, sl]
                        g = 1.0 / (1.0 + jnp.exp(-t))
                        gate_b[b, r, sl] = g
                        msg_b[r, sl] = sm_g[b, r, pl.ds(H + k * 16, 16)] * g
                return 0
            lax.fori_loop(0, C // 2, comp, 0)

            pltpu.sync_copy(gate_b.at[b], acc_g.at[dst_idx.at[b]], add=True)
            pltpu.sync_copy(msg_b, acc_m.at[dst_idx.at[b]], add=True)
            pltpu.async_copy(gate_b.at[b], gate_out.at[pl.ds(base, C)], gws[b])

        issue_idx(0, tile_base)
        issue_idx(1, tile_base + C)
        issue_gather(0, tile_base)

        def step(i, _):
            b = lax.rem(i, 2)
            nxt = tile_base + lax.rem(i + 1, NCHUNK) * C

            @pl.when(b == 0)
            def _():
                issue_gather(1, nxt)
                do_chunk(0, i, tile_base + i * C)
                issue_idx(0, tile_base + lax.rem(i + 2, NCHUNK) * C)

            @pl.when(b == 1)
            def _():
                issue_gather(0, nxt)
                do_chunk(1, i, tile_base + i * C)
                issue_idx(1, tile_base + lax.rem(i + 2, NCHUNK) * C)
            return 0
        lax.fori_loop(0, NCHUNK, step, 0)

        wait_in(0)
        pltpu.make_async_copy(
            src.at[pl.ds(0, C)], src_idx.at[1], isems[1]).wait()
        pltpu.make_async_copy(
            dst.at[pl.ds(0, C)], dst_idx.at[1], isems[1]).wait()
        pltpu.make_async_copy(
            gate_b.at[0], gate_out.at[pl.ds(0, C)], gws[0]).wait()
        pltpu.make_async_copy(
            gate_b.at[1], gate_out.at[pl.ds(0, C)], gws[1]).wait()

        plsc.subcore_barrier()
        pltpu.sync_copy(acc_g.at[pl.ds(row0, NPT)], gs_out.at[pl.ds(row0, NPT)])
        pltpu.sync_copy(acc_m.at[pl.ds(row0, NPT)], ms_out.at[pl.ds(row0, NPT)])

    @pl.when(cid == 0)
    def _():
        run_half(tsm_lo, xd_lo, ea_lo, gate_lo, gs_lo, ms_lo)

    @pl.when(cid == 1)
    def _():
        run_half(tsm_hi, xd_hi, ea_hi, gate_hi, gs_hi, ms_hi)


def _sc_gate(src, dst, tsm_lo, tsm_hi, xd_lo, xd_hi, ea_lo, ea_hi):
    mesh = plsc.VectorSubcoreMesh(core_axis_name="c", subcore_axis_name="s")
    f = pl.kernel(
        _sc_gate_body,
        out_type=[jax.ShapeDtypeStruct((E, H), _F32)] * 2
        + [jax.ShapeDtypeStruct((NPAD, H), _F32)] * 4,
        mesh=mesh,
        scratch_types=[
            pltpu.VMEM((2, C), jnp.int32),
            pltpu.VMEM((2, C), jnp.int32),
            pltpu.VMEM((2, C, D), _F32),
            pltpu.VMEM((2, C, H), _F32),
            pltpu.VMEM((2, C, H), _F32),
            pltpu.VMEM((2, C, H), _F32),
            pltpu.VMEM((C, H), _F32),
            pltpu.VMEM_SHARED((NPAD, H), _F32),
            pltpu.VMEM_SHARED((NPAD, H), _F32),
            pltpu.SemaphoreType.DMA,
            pltpu.SemaphoreType.DMA,
            pltpu.SemaphoreType.DMA,
            pltpu.SemaphoreType.DMA,
            pltpu.SemaphoreType.DMA,
            pltpu.SemaphoreType.DMA,
        ],
        compiler_params=pltpu.CompilerParams(use_tc_tiling_on_sc=False),
    )
    return f(src, dst, tsm_lo, tsm_hi, xd_lo, xd_hi, ea_lo, ea_hi)


def _edge_stats_body(glo_ref, ghi_ref, w_ref, b_ref, stats_ref, acc_ref):
    i = pl.program_id(0)

    @pl.when(i == 0)
    def _():
        acc_ref[...] = jnp.zeros_like(acc_ref)

    w = w_ref[...]
    h = (jnp.dot(glo_ref[...], w[:H, :], preferred_element_type=_F32)
         + jnp.dot(ghi_ref[...], w[H:, :], preferred_element_type=_F32)
         + b_ref[...])
    acc_ref[0:1, :] += jnp.sum(h, axis=0, keepdims=True)
    acc_ref[1:2, :] += jnp.sum(h * h, axis=0, keepdims=True)

    @pl.when(i == pl.num_programs(0) - 1)
    def _():
        stats_ref[...] = acc_ref[...]


def _edge_stats(gate_lo, gate_hi, W_eo, beo):
    bn = 4000
    grid = (E // bn,)
    return pl.pallas_call(
        _edge_stats_body,
        grid=grid,
        in_specs=[pl.BlockSpec((bn, H), lambda i: (i, 0)),
                  pl.BlockSpec((bn, H), lambda i: (i, 0)),
                  pl.BlockSpec((D, D), lambda i: (0, 0)),
                  pl.BlockSpec((1, D), lambda i: (0, 0))],
        out_specs=pl.BlockSpec((8, D), lambda i: (0, 0)),
        out_shape=jax.ShapeDtypeStruct((8, D), _F32),
        scratch_shapes=[pltpu.VMEM((8, D), _F32)],
    )(gate_lo, gate_hi, W_eo, beo)


def _edge_out_body(glo_ref, ghi_ref, w_ref, b_ref, stats_ref, gam_ref,
                   bt_ref, out_ref):
    w = w_ref[...]
    h = (jnp.dot(glo_ref[...], w[:H, :], preferred_element_type=_F32)
         + jnp.dot(ghi_ref[...], w[H:, :], preferred_element_type=_F32)
         + b_ref[...])
    mu = stats_ref[0:1, :] / E
    var = stats_ref[1:2, :] / E - mu * mu
    scale = gam_ref[...] * lax.rsqrt(var + 1e-5)
    out_ref[...] = jnp.maximum((h - mu) * scale + bt_ref[...], 0.0)


def _edge_out(gate_lo, gate_hi, W_eo, beo, stats, ge, bte):
    bn = 4000
    grid = (E // bn,)
    return pl.pallas_call(
        _edge_out_body,
        grid=grid,
        in_specs=[pl.BlockSpec((bn, H), lambda i: (i, 0)),
                  pl.BlockSpec((bn, H), lambda i: (i, 0)),
                  pl.BlockSpec((D, D), lambda i: (0, 0)),
                  pl.BlockSpec((1, D), lambda i: (0, 0)),
                  pl.BlockSpec((8, D), lambda i: (0, 0)),
                  pl.BlockSpec((1, D), lambda i: (0, 0)),
                  pl.BlockSpec((1, D), lambda i: (0, 0))],
        out_specs=pl.BlockSpec((bn, D), lambda i: (i, 0)),
        out_shape=jax.ShapeDtypeStruct((E, D), _F32),
    )(gate_lo, gate_hi, W_eo, beo, stats, ge, bte)


def _node_out_body(xr_ref, gslo_ref, gshi_ref, mslo_ref, mshi_ref,
                   gam_ref, bt_ref, out_ref):
    agg_lo = mslo_ref[...] / (gslo_ref[...] + 1e-6)
    agg_hi = mshi_ref[...] / (gshi_ref[...] + 1e-6)
    t = xr_ref[...] + jnp.concatenate([agg_lo, agg_hi], axis=1)
    mu = jnp.mean(t, axis=0, keepdims=True)
    var = jnp.mean((t - mu) ** 2, axis=0, keepdims=True)
    norm = gam_ref[...] * (t - mu) * lax.rsqrt(var + 1e-5) + bt_ref[...]
    out_ref[...] = jnp.maximum(norm, 0.0)


def _node_out(xr, gs_lo, gs_hi, ms_lo, ms_hi, gn, btn):
    half = pl.BlockSpec((N, H), lambda i: (0, 0))
    return pl.pallas_call(
        _node_out_body,
        grid=(1,),
        in_specs=[pl.BlockSpec((N, D), lambda i: (0, 0)),
                  half, half, half, half,
                  pl.BlockSpec((1, D), lambda i: (0, 0)),
                  pl.BlockSpec((1, D), lambda i: (0, 0))],
        out_specs=pl.BlockSpec((N, D), lambda i: (0, 0)),
        out_shape=jax.ShapeDtypeStruct((N, D), _F32),
    )(xr, gs_lo, gs_hi, ms_lo, ms_hi, gn, btn)


@jax.jit
def kernel(x, edge_index, edge_attr, W_src, b_src, W_dst, b_dst, W_edge,
           b_edge, W_msg, b_msg, W_res, b_res, W_eo, b_eo, gamma_n, beta_n,
           gamma_e, beta_e):
    r = lambda b: b.reshape(1, D)
    tsm_lo, tsm_hi, xd_lo, xd_hi, xr = _node_fwd(
        x, W_src, W_dst, W_msg, W_res, r(b_src), r(b_dst), r(b_msg), r(b_res))
    ea_lo, ea_hi = _edge_fwd(edge_attr, W_edge, r(b_edge))
    gate_lo, gate_hi, gs_lo, gs_hi, ms_lo, ms_hi = _sc_gate(
        edge_index[0], edge_index[1], tsm_lo, tsm_hi, xd_lo, xd_hi,
        ea_lo, ea_hi)
    stats = _edge_stats(gate_lo, gate_hi, W_eo, r(b_eo))
    edge_new = _edge_out(gate_lo, gate_hi, W_eo, r(b_eo), stats,
                         r(gamma_e), r(beta_e))
    x_out = _node_out(xr, gs_lo, gs_hi, ms_lo, ms_hi, r(gamma_n), r(beta_n))
    return (x_out, edge_new)


# speedup vs baseline: 1.1945x; 1.0445x over previous
import jax
import jax.numpy as jnp
from jax import lax
from jax.experimental import pallas as pl
from jax.experimental.pallas import tpu as pltpu
from jax.experimental.pallas import tpu_sc as plsc

N = 10000
E = 320000
D = 128
H = D // 2

NUM_TILES = 16
EPT = E // NUM_TILES
C = 40
NCHUNK = EPT // C
NPAD = 10240
NPT = NPAD // NUM_TILES

_F32 = jnp.float32


def _node_fwd_body(x_ref, ws_ref, wd_ref, wm_ref, wr_ref,
                   bs_ref, bd_ref, bm_ref, br_ref,
                   tsm_lo, tsm_hi, xd_lo, xd_hi, xr_ref):
    xb = x_ref[...]

    def lin(w_ref, b_ref):
        return jnp.dot(xb, w_ref[...], preferred_element_type=_F32) + b_ref[...]

    xs = lin(ws_ref, bs_ref)
    xm = lin(wm_ref, bm_ref)
    tsm_lo[...] = jnp.concatenate([xs[:, :H], xm[:, :H]], axis=1)
    tsm_hi[...] = jnp.concatenate([xs[:, H:], xm[:, H:]], axis=1)
    xd = lin(wd_ref, bd_ref)
    xd_lo[...] = xd[:, :H]
    xd_hi[...] = xd[:, H:]
    xr_ref[...] = lin(wr_ref, br_ref)


def _node_fwd(x, W_src, W_dst, W_msg, W_res, bs, bd, bm, br):
    bn = 1000
    grid = (N // bn,)
    w_spec = pl.BlockSpec((D, D), lambda i: (0, 0))
    b_spec = pl.BlockSpec((1, D), lambda i: (0, 0))
    full = pl.BlockSpec((bn, D), lambda i: (i, 0))
    half = pl.BlockSpec((bn, H), lambda i: (i, 0))
    return pl.pallas_call(
        _node_fwd_body,
        grid=grid,
        in_specs=[full, w_spec, w_spec, w_spec, w_spec,
                  b_spec, b_spec, b_spec, b_spec],
        out_specs=[full, full, half, half, full],
        out_shape=[jax.ShapeDtypeStruct((N, D), _F32)] * 2
        + [jax.ShapeDtypeStruct((N, H), _F32)] * 2
        + [jax.ShapeDtypeStruct((N, D), _F32)],
    )(x, W_src, W_dst, W_msg, W_res, bs, bd, bm, br)


def _edge_fwd_body(a_ref, w_ref, b_ref, lo_ref, hi_ref):
    ea = (jnp.dot(a_ref[...], w_ref[...], preferred_element_type=_F32)
          + b_ref[...])
    lo_ref[...] = ea[:, :H]
    hi_ref[...] = ea[:, H:]


def _edge_fwd(edge_attr, W_edge, be):
    bn = 4000
    grid = (E // bn,)
    return pl.pallas_call(
        _edge_fwd_body,
        grid=grid,
        in_specs=[pl.BlockSpec((bn, D), lambda i: (i, 0)),
                  pl.BlockSpec((D, D), lambda i: (0, 0)),
                  pl.BlockSpec((1, D), lambda i: (0, 0))],
        out_specs=[pl.BlockSpec((bn, H), lambda i: (i, 0)),
                   pl.BlockSpec((bn, H), lambda i: (i, 0))],
        out_shape=[jax.ShapeDtypeStruct((E, H), _F32)] * 2,
    )(edge_attr, W_edge, be)


def _sc_gate_body(src, dst, tsm_lo, tsm_hi, xd_lo, xd_hi, ea_lo, ea_hi,
                  gate_lo, gate_hi, gs_lo, gs_hi, ms_lo, ms_hi,
                  src_idx, dst_idx, sm_g, xd_g, ea_b, gate_b, msg_b,
                  acc_g, acc_m, sem0, sem1, gw0, gw1,
                  isem0, isem1, isem2, isem3, ssem0, ssem1):
    cid = lax.axis_index("c")
    sid = lax.axis_index("s")

    def run_half(tsm_t, xd_t, ea_t, gate_out, gs_out, ms_out):
        sems = (sem0, sem1)
        gws = (gw0, gw1)
        isems = (isem0, isem1, isem2, isem3)
        ssems = (ssem0, ssem1)
        tile_base = sid * EPT

        def zfill(r, _):
            for k in range(H // 16):
                msg_b[0, r, pl.ds(k * 16, 16)] = jnp.zeros((16,), _F32)
            return 0
        lax.fori_loop(0, C, zfill, 0)
        row0 = sid * NPT

        def zcopy(j, _):
            pltpu.sync_copy(msg_b.at[0], acc_g.at[pl.ds(row0 + j * C, C)])
            pltpu.sync_copy(msg_b.at[0], acc_m.at[pl.ds(row0 + j * C, C)])
            return 0
        lax.fori_loop(0, NPT // C, zcopy, 0)
        plsc.subcore_barrier()

        def issue_idx(ib, base):
            pltpu.async_copy(src.at[pl.ds(base, C)], src_idx.at[ib], isems[ib])
            pltpu.async_copy(dst.at[pl.ds(base, C)], dst_idx.at[ib], isems[ib])

        def issue_gather(b, ib, base):
            pltpu.make_async_copy(
                src.at[pl.ds(0, C)], src_idx.at[ib], isems[ib]).wait()
            pltpu.make_async_copy(
                dst.at[pl.ds(0, C)], dst_idx.at[ib], isems[ib]).wait()
            pltpu.async_copy(tsm_t.at[src_idx.at[ib]], sm_g.at[b], sems[b])
            pltpu.async_copy(xd_t.at[dst_idx.at[ib]], xd_g.at[b], sems[b])
            pltpu.async_copy(ea_t.at[pl.ds(base, C)], ea_b.at[b], sems[b])

        def wait_in(b, ib):
            pltpu.make_async_copy(
                tsm_t.at[src_idx.at[ib]], sm_g.at[b], sems[b]).wait()
            pltpu.make_async_copy(
                xd_t.at[dst_idx.at[ib]], xd_g.at[b], sems[b]).wait()
            pltpu.make_async_copy(
                ea_t.at[pl.ds(0, C)], ea_b.at[b], sems[b]).wait()

        def wait_scatters(b, ib):
            pltpu.make_async_copy(
                gate_b.at[b], acc_g.at[dst_idx.at[ib]], ssems[b]).wait()
            pltpu.make_async_copy(
                msg_b.at[b], acc_m.at[dst_idx.at[ib]], ssems[b]).wait()

        def do_chunk(b, ib, i, base):
            wait_in(b, ib)

            @pl.when(i >= 2)
            def _():
                pltpu.make_async_copy(
                    gate_b.at[b], gate_out.at[pl.ds(0, C)], gws[b]).wait()
                wait_scatters(b, ib)

            def comp(rr, _):
                for r in (2 * rr, 2 * rr + 1):
                    for k in range(H // 16):
                        sl = pl.ds(k * 16, 16)
                        t = sm_g[b, r, sl] + xd_g[b, r, sl] + ea_b[b, r, sl]
                        g = 1.0 / (1.0 + jnp.exp(-t))
                        gate_b[b, r, sl] = g
                        msg_b[b, r, sl] = sm_g[b, r, pl.ds(H + k * 16, 16)] * g
                return 0
            lax.fori_loop(0, C // 2, comp, 0)

            pltpu.async_copy(gate_b.at[b], acc_g.at[dst_idx.at[ib]],
                             ssems[b], add=True)
            pltpu.async_copy(msg_b.at[b], acc_m.at[dst_idx.at[ib]],
                             ssems[b], add=True)
            pltpu.async_copy(gate_b.at[b], gate_out.at[pl.ds(base, C)], gws[b])

        issue_idx(0, tile_base)
        issue_idx(1, tile_base + C)
        issue_gather(0, 0, tile_base)

        def step(i, _):
            m = lax.rem(i, 4)
            for k in range(4):
                @pl.when(m == k)
                def _(k=k):
                    issue_gather((k + 1) % 2, (k + 1) % 4,
                                 tile_base + lax.rem(i + 1, NCHUNK) * C)
                    do_chunk(k % 2, k, i, tile_base + i * C)
                    issue_idx((k + 2) % 4,
                              tile_base + lax.rem(i + 2, NCHUNK) * C)
            return 0
        lax.fori_loop(0, NCHUNK, step, 0)

        wait_in(0, 0)
        pltpu.make_async_copy(
            src.at[pl.ds(0, C)], src_idx.at[1], isems[1]).wait()
        pltpu.make_async_copy(
            dst.at[pl.ds(0, C)], dst_idx.at[1], isems[1]).wait()
        pltpu.make_async_copy(
            gate_b.at[0], gate_out.at[pl.ds(0, C)], gws[0]).wait()
        pltpu.make_async_copy(
            gate_b.at[1], gate_out.at[pl.ds(0, C)], gws[1]).wait()
        wait_scatters(0, 2)
        wait_scatters(1, 3)

        plsc.subcore_barrier()
        pltpu.sync_copy(acc_g.at[pl.ds(row0, NPT)], gs_out.at[pl.ds(row0, NPT)])
        pltpu.sync_copy(acc_m.at[pl.ds(row0, NPT)], ms_out.at[pl.ds(row0, NPT)])

    @pl.when(cid == 0)
    def _():
        run_half(tsm_lo, xd_lo, ea_lo, gate_lo, gs_lo, ms_lo)

    @pl.when(cid == 1)
    def _():
        run_half(tsm_hi, xd_hi, ea_hi, gate_hi, gs_hi, ms_hi)


def _sc_gate(src, dst, tsm_lo, tsm_hi, xd_lo, xd_hi, ea_lo, ea_hi):
    mesh = plsc.VectorSubcoreMesh(core_axis_name="c", subcore_axis_name="s")
    f = pl.kernel(
        _sc_gate_body,
        out_type=[jax.ShapeDtypeStruct((E, H), _F32)] * 2
        + [jax.ShapeDtypeStruct((NPAD, H), _F32)] * 4,
        mesh=mesh,
        scratch_types=[
            pltpu.VMEM((4, C), jnp.int32),
            pltpu.VMEM((4, C), jnp.int32),
            pltpu.VMEM((2, C, D), _F32),
            pltpu.VMEM((2, C, H), _F32),
            pltpu.VMEM((2, C, H), _F32),
            pltpu.VMEM((2, C, H), _F32),
            pltpu.VMEM((2, C, H), _F32),
            pltpu.VMEM_SHARED((NPAD, H), _F32),
            pltpu.VMEM_SHARED((NPAD, H), _F32),
            pltpu.SemaphoreType.DMA,
            pltpu.SemaphoreType.DMA,
            pltpu.SemaphoreType.DMA,
            pltpu.SemaphoreType.DMA,
            pltpu.SemaphoreType.DMA,
            pltpu.SemaphoreType.DMA,
            pltpu.SemaphoreType.DMA,
            pltpu.SemaphoreType.DMA,
            pltpu.SemaphoreType.DMA,
            pltpu.SemaphoreType.DMA,
        ],
        compiler_params=pltpu.CompilerParams(use_tc_tiling_on_sc=False),
    )
    return f(src, dst, tsm_lo, tsm_hi, xd_lo, xd_hi, ea_lo, ea_hi)


def _edge_stats_body(glo_ref, ghi_ref, w_ref, b_ref, stats_ref, acc_ref):
    i = pl.program_id(0)

    @pl.when(i == 0)
    def _():
        acc_ref[...] = jnp.zeros_like(acc_ref)

    w = w_ref[...]
    h = (jnp.dot(glo_ref[...], w[:H, :], preferred_element_type=_F32)
         + jnp.dot(ghi_ref[...], w[H:, :], preferred_element_type=_F32)
         + b_ref[...])
    acc_ref[0:1, :] += jnp.sum(h, axis=0, keepdims=True)
    acc_ref[1:2, :] += jnp.sum(h * h, axis=0, keepdims=True)

    @pl.when(i == pl.num_programs(0) - 1)
    def _():
        stats_ref[...] = acc_ref[...]


def _edge_stats(gate_lo, gate_hi, W_eo, beo):
    bn = 4000
    grid = (E // bn,)
    return pl.pallas_call(
        _edge_stats_body,
        grid=grid,
        in_specs=[pl.BlockSpec((bn, H), lambda i: (i, 0)),
                  pl.BlockSpec((bn, H), lambda i: (i, 0)),
                  pl.BlockSpec((D, D), lambda i: (0, 0)),
                  pl.BlockSpec((1, D), lambda i: (0, 0))],
        out_specs=pl.BlockSpec((8, D), lambda i: (0, 0)),
        out_shape=jax.ShapeDtypeStruct((8, D), _F32),
        scratch_shapes=[pltpu.VMEM((8, D), _F32)],
    )(gate_lo, gate_hi, W_eo, beo)


def _edge_out_body(glo_ref, ghi_ref, w_ref, b_ref, stats_ref, gam_ref,
                   bt_ref, out_ref):
    w = w_ref[...]
    h = (jnp.dot(glo_ref[...], w[:H, :], preferred_element_type=_F32)
         + jnp.dot(ghi_ref[...], w[H:, :], preferred_element_type=_F32)
         + b_ref[...])
    mu = stats_ref[0:1, :] / E
    var = stats_ref[1:2, :] / E - mu * mu
    scale = gam_ref[...] * lax.rsqrt(var + 1e-5)
    out_ref[...] = jnp.maximum((h - mu) * scale + bt_ref[...], 0.0)


def _edge_out(gate_lo, gate_hi, W_eo, beo, stats, ge, bte):
    bn = 4000
    grid = (E // bn,)
    return pl.pallas_call(
        _edge_out_body,
        grid=grid,
        in_specs=[pl.BlockSpec((bn, H), lambda i: (i, 0)),
                  pl.BlockSpec((bn, H), lambda i: (i, 0)),
                  pl.BlockSpec((D, D), lambda i: (0, 0)),
                  pl.BlockSpec((1, D), lambda i: (0, 0)),
                  pl.BlockSpec((8, D), lambda i: (0, 0)),
                  pl.BlockSpec((1, D), lambda i: (0, 0)),
                  pl.BlockSpec((1, D), lambda i: (0, 0))],
        out_specs=pl.BlockSpec((bn, D), lambda i: (i, 0)),
        out_shape=jax.ShapeDtypeStruct((E, D), _F32),
    )(gate_lo, gate_hi, W_eo, beo, stats, ge, bte)


def _node_out_body(xr_ref, gslo_ref, gshi_ref, mslo_ref, mshi_ref,
                   gam_ref, bt_ref, out_ref):
    agg_lo = mslo_ref[...] / (gslo_ref[...] + 1e-6)
    agg_hi = mshi_ref[...] / (gshi_ref[...] + 1e-6)
    t = xr_ref[...] + jnp.concatenate([agg_lo, agg_hi], axis=1)
    mu = jnp.mean(t, axis=0, keepdims=True)
    var = jnp.mean((t - mu) ** 2, axis=0, keepdims=True)
    norm = gam_ref[...] * (t - mu) * lax.rsqrt(var + 1e-5) + bt_ref[...]
    out_ref[...] = jnp.maximum(norm, 0.0)


def _node_out(xr, gs_lo, gs_hi, ms_lo, ms_hi, gn, btn):
    half = pl.BlockSpec((N, H), lambda i: (0, 0))
    return pl.pallas_call(
        _node_out_body,
        grid=(1,),
        in_specs=[pl.BlockSpec((N, D), lambda i: (0, 0)),
                  half, half, half, half,
                  pl.BlockSpec((1, D), lambda i: (0, 0)),
                  pl.BlockSpec((1, D), lambda i: (0, 0))],
        out_specs=pl.BlockSpec((N, D), lambda i: (0, 0)),
        out_shape=jax.ShapeDtypeStruct((N, D), _F32),
    )(xr, gs_lo, gs_hi, ms_lo, ms_hi, gn, btn)


@jax.jit
def kernel(x, edge_index, edge_attr, W_src, b_src, W_dst, b_dst, W_edge,
           b_edge, W_msg, b_msg, W_res, b_res, W_eo, b_eo, gamma_n, beta_n,
           gamma_e, beta_e):
    r = lambda b: b.reshape(1, D)
    tsm_lo, tsm_hi, xd_lo, xd_hi, xr = _node_fwd(
        x, W_src, W_dst, W_msg, W_res, r(b_src), r(b_dst), r(b_msg), r(b_res))
    ea_lo, ea_hi = _edge_fwd(edge_attr, W_edge, r(b_edge))
    gate_lo, gate_hi, gs_lo, gs_hi, ms_lo, ms_hi = _sc_gate(
        edge_index[0], edge_index[1], tsm_lo, tsm_hi, xd_lo, xd_hi,
        ea_lo, ea_hi)
    stats = _edge_stats(gate_lo, gate_hi, W_eo, r(b_eo))
    edge_new = _edge_out(gate_lo, gate_hi, W_eo, r(b_eo), stats,
                         r(gamma_e), r(beta_e))
    x_out = _node_out(xr, gs_lo, gs_hi, ms_lo, ms_hi, r(gamma_n), r(beta_n))
    return (x_out, edge_new)


# speedup vs baseline: 2.4024x; 2.0113x over previous
import jax
import jax.numpy as jnp
from jax import lax
from jax.experimental import pallas as pl
from jax.experimental.pallas import tpu as pltpu
from jax.experimental.pallas import tpu_sc as plsc

N = 10000
E = 320000
D = 128
H = D // 2

NUM_TILES = 16
EPT = E // NUM_TILES
C = 40
NCHUNK = EPT // C
NPAD = 10240
NPT = NPAD // NUM_TILES

_F32 = jnp.float32


def _node_fwd_body(x_ref, ws_ref, wd_ref, wm_ref, wr_ref,
                   bs_ref, bd_ref, bm_ref, br_ref,
                   tsm_lo, tsm_hi, xd_lo, xd_hi, xr_ref):
    xb = x_ref[...]

    def lin(w_ref, b_ref):
        return jnp.dot(xb, w_ref[...], preferred_element_type=_F32) + b_ref[...]

    xs = lin(ws_ref, bs_ref)
    xm = lin(wm_ref, bm_ref)
    tsm_lo[...] = jnp.concatenate([xs[:, :H], xm[:, :H]], axis=1)
    tsm_hi[...] = jnp.concatenate([xs[:, H:], xm[:, H:]], axis=1)
    xd = lin(wd_ref, bd_ref)
    xd_lo[...] = xd[:, :H]
    xd_hi[...] = xd[:, H:]
    xr_ref[...] = lin(wr_ref, br_ref)


def _node_fwd(x, W_src, W_dst, W_msg, W_res, bs, bd, bm, br):
    bn = 1000
    grid = (N // bn,)
    w_spec = pl.BlockSpec((D, D), lambda i: (0, 0))
    b_spec = pl.BlockSpec((1, D), lambda i: (0, 0))
    full = pl.BlockSpec((bn, D), lambda i: (i, 0))
    half = pl.BlockSpec((bn, H), lambda i: (i, 0))
    return pl.pallas_call(
        _node_fwd_body,
        grid=grid,
        in_specs=[full, w_spec, w_spec, w_spec, w_spec,
                  b_spec, b_spec, b_spec, b_spec],
        out_specs=[full, full, half, half, full],
        out_shape=[jax.ShapeDtypeStruct((N, D), _F32)] * 2
        + [jax.ShapeDtypeStruct((N, H), _F32)] * 2
        + [jax.ShapeDtypeStruct((N, D), _F32)],
    )(x, W_src, W_dst, W_msg, W_res, bs, bd, bm, br)


def _edge_fwd_body(a_ref, w_ref, b_ref, lo_ref, hi_ref):
    ea = (jnp.dot(a_ref[...], w_ref[...], preferred_element_type=_F32)
          + b_ref[...])
    lo_ref[...] = ea[:, :H]
    hi_ref[...] = ea[:, H:]


def _edge_fwd(edge_attr, W_edge, be):
    bn = 4000
    grid = (E // bn,)
    return pl.pallas_call(
        _edge_fwd_body,
        grid=grid,
        in_specs=[pl.BlockSpec((bn, D), lambda i: (i, 0)),
                  pl.BlockSpec((D, D), lambda i: (0, 0)),
                  pl.BlockSpec((1, D), lambda i: (0, 0))],
        out_specs=[pl.BlockSpec((bn, H), lambda i: (i, 0)),
                   pl.BlockSpec((bn, H), lambda i: (i, 0))],
        out_shape=[jax.ShapeDtypeStruct((E, H), _F32)] * 2,
    )(edge_attr, W_edge, be)


def _sc_gate_body(src, dst, tsm_lo, tsm_hi, xd_lo, xd_hi, ea_lo, ea_hi,
                  gate_lo, gate_hi, gs_lo, gs_hi, ms_lo, ms_hi,
                  src_idx, dst_idx, sm_g, xd_g, ea_b, gate_b, msg_b,
                  acc_g, acc_m, sem0, sem1, gw0, gw1,
                  isem0, isem1, isem2, isem3, ssem0, ssem1):
    cid = lax.axis_index("c")
    sid = lax.axis_index("s")

    def run_half(tsm_t, xd_t, ea_t, gate_out, gs_out, ms_out):
        sems = (sem0, sem1)
        gws = (gw0, gw1)
        isems = (isem0, isem1, isem2, isem3)
        ssems = (ssem0, ssem1)
        tile_base = sid * EPT

        def zfill(r, _):
            for k in range(H // 16):
                msg_b[0, r, pl.ds(k * 16, 16)] = jnp.zeros((16,), _F32)
            return 0
        lax.fori_loop(0, C, zfill, 0)
        row0 = sid * NPT

        def zcopy(j, _):
            pltpu.sync_copy(msg_b.at[0], acc_g.at[pl.ds(row0 + j * C, C)])
            pltpu.sync_copy(msg_b.at[0], acc_m.at[pl.ds(row0 + j * C, C)])
            return 0
        lax.fori_loop(0, NPT // C, zcopy, 0)
        plsc.subcore_barrier()

        def issue_idx(ib, base):
            pltpu.async_copy(src.at[pl.ds(base, C)], src_idx.at[ib], isems[ib])
            pltpu.async_copy(dst.at[pl.ds(base, C)], dst_idx.at[ib], isems[ib])

        def issue_gather(b, ib, base):
            pltpu.make_async_copy(
                src.at[pl.ds(0, C)], src_idx.at[ib], isems[ib]).wait()
            pltpu.make_async_copy(
                dst.at[pl.ds(0, C)], dst_idx.at[ib], isems[ib]).wait()
            pltpu.async_copy(tsm_t.at[src_idx.at[ib]], sm_g.at[b], sems[b])
            pltpu.async_copy(xd_t.at[dst_idx.at[ib]], xd_g.at[b], sems[b])
            pltpu.async_copy(ea_t.at[pl.ds(base, C)], ea_b.at[b], sems[b])

        def wait_in(b, ib):
            pltpu.make_async_copy(
                tsm_t.at[src_idx.at[ib]], sm_g.at[b], sems[b]).wait()
            pltpu.make_async_copy(
                xd_t.at[dst_idx.at[ib]], xd_g.at[b], sems[b]).wait()
            pltpu.make_async_copy(
                ea_t.at[pl.ds(0, C)], ea_b.at[b], sems[b]).wait()

        def wait_scatters(b, ib):
            pltpu.make_async_copy(
                gate_b.at[b], acc_g.at[dst_idx.at[ib]], ssems[b]).wait()
            pltpu.make_async_copy(
                msg_b.at[b], acc_m.at[dst_idx.at[ib]], ssems[b]).wait()

        def do_chunk(b, ib, i, base):
            wait_in(b, ib)

            @pl.when(i >= 2)
            def _():
                pltpu.make_async_copy(
                    gate_b.at[b], gate_out.at[pl.ds(0, C)], gws[b]).wait()
                wait_scatters(b, ib)

            def comp(rr, _):
                rk = [(r, k) for r in (2 * rr, 2 * rr + 1)
                      for k in range(H // 16)]
                ts = [sm_g[b, r, pl.ds(k * 16, 16)]
                      + xd_g[b, r, pl.ds(k * 16, 16)]
                      + ea_b[b, r, pl.ds(k * 16, 16)] for r, k in rk]
                es = [jnp.exp(-t) for t in ts]
                gs = [1.0 / (1.0 + e) for e in es]
                ms = [sm_g[b, r, pl.ds(H + k * 16, 16)] * g
                      for (r, k), g in zip(rk, gs)]
                for (r, k), g, m in zip(rk, gs, ms):
                    gate_b[b, r, pl.ds(k * 16, 16)] = g
                    msg_b[b, r, pl.ds(k * 16, 16)] = m
                return 0
            lax.fori_loop(0, C // 2, comp, 0)

            pltpu.async_copy(gate_b.at[b], acc_g.at[dst_idx.at[ib]],
                             ssems[b], add=True)
            pltpu.async_copy(msg_b.at[b], acc_m.at[dst_idx.at[ib]],
                             ssems[b], add=True)
            pltpu.async_copy(gate_b.at[b], gate_out.at[pl.ds(base, C)], gws[b])

        issue_idx(0, tile_base)
        issue_idx(1, tile_base + C)
        issue_gather(0, 0, tile_base)

        def step(i, _):
            m = lax.rem(i, 4)
            for k in range(4):
                @pl.when(m == k)
                def _(k=k):
                    issue_gather((k + 1) % 2, (k + 1) % 4,
                                 tile_base + lax.rem(i + 1, NCHUNK) * C)
                    do_chunk(k % 2, k, i, tile_base + i * C)
                    issue_idx((k + 2) % 4,
                              tile_base + lax.rem(i + 2, NCHUNK) * C)
            return 0
        lax.fori_loop(0, NCHUNK, step, 0)

        wait_in(0, 0)
        pltpu.make_async_copy(
            src.at[pl.ds(0, C)], src_idx.at[1], isems[1]).wait()
        pltpu.make_async_copy(
            dst.at[pl.ds(0, C)], dst_idx.at[1], isems[1]).wait()
        pltpu.make_async_copy(
            gate_b.at[0], gate_out.at[pl.ds(0, C)], gws[0]).wait()
        pltpu.make_async_copy(
            gate_b.at[1], gate_out.at[pl.ds(0, C)], gws[1]).wait()
        wait_scatters(0, 2)
        wait_scatters(1, 3)

        plsc.subcore_barrier()
        pltpu.sync_copy(acc_g.at[pl.ds(row0, NPT)], gs_out.at[pl.ds(row0, NPT)])
        pltpu.sync_copy(acc_m.at[pl.ds(row0, NPT)], ms_out.at[pl.ds(row0, NPT)])

    @pl.when(cid == 0)
    def _():
        run_half(tsm_lo, xd_lo, ea_lo, gate_lo, gs_lo, ms_lo)

    @pl.when(cid == 1)
    def _():
        run_half(tsm_hi, xd_hi, ea_hi, gate_hi, gs_hi, ms_hi)


def _sc_gate(src, dst, tsm_lo, tsm_hi, xd_lo, xd_hi, ea_lo, ea_hi):
    mesh = plsc.VectorSubcoreMesh(core_axis_name="c", subcore_axis_name="s")
    f = pl.kernel(
        _sc_gate_body,
        out_type=[jax.ShapeDtypeStruct((E, H), _F32)] * 2
        + [jax.ShapeDtypeStruct((NPAD, H), _F32)] * 4,
        mesh=mesh,
        scratch_types=[
            pltpu.VMEM((4, C), jnp.int32),
            pltpu.VMEM((4, C), jnp.int32),
            pltpu.VMEM((2, C, D), _F32),
            pltpu.VMEM((2, C, H), _F32),
            pltpu.VMEM((2, C, H), _F32),
            pltpu.VMEM((2, C, H), _F32),
            pltpu.VMEM((2, C, H), _F32),
            pltpu.VMEM_SHARED((NPAD, H), _F32),
            pltpu.VMEM_SHARED((NPAD, H), _F32),
            pltpu.SemaphoreType.DMA,
            pltpu.SemaphoreType.DMA,
            pltpu.SemaphoreType.DMA,
            pltpu.SemaphoreType.DMA,
            pltpu.SemaphoreType.DMA,
            pltpu.SemaphoreType.DMA,
            pltpu.SemaphoreType.DMA,
            pltpu.SemaphoreType.DMA,
            pltpu.SemaphoreType.DMA,
            pltpu.SemaphoreType.DMA,
        ],
        compiler_params=pltpu.CompilerParams(use_tc_tiling_on_sc=False),
    )
    return f(src, dst, tsm_lo, tsm_hi, xd_lo, xd_hi, ea_lo, ea_hi)


def _edge_stats_body(glo_ref, ghi_ref, w_ref, b_ref, stats_ref, acc_ref):
    i = pl.program_id(0)

    @pl.when(i == 0)
    def _():
        acc_ref[...] = jnp.zeros_like(acc_ref)

    w = w_ref[...]
    h = (jnp.dot(glo_ref[...], w[:H, :], preferred_element_type=_F32)
         + jnp.dot(ghi_ref[...], w[H:, :], preferred_element_type=_F32)
         + b_ref[...])
    acc_ref[0:1, :] += jnp.sum(h, axis=0, keepdims=True)
    acc_ref[1:2, :] += jnp.sum(h * h, axis=0, keepdims=True)

    @pl.when(i == pl.num_programs(0) - 1)
    def _():
        stats_ref[...] = acc_ref[...]


def _edge_stats(gate_lo, gate_hi, W_eo, beo):
    bn = 4000
    grid = (E // bn,)
    return pl.pallas_call(
        _edge_stats_body,
        grid=grid,
        in_specs=[pl.BlockSpec((bn, H), lambda i: (i, 0)),
                  pl.BlockSpec((bn, H), lambda i: (i, 0)),
                  pl.BlockSpec((D, D), lambda i: (0, 0)),
                  pl.BlockSpec((1, D), lambda i: (0, 0))],
        out_specs=pl.BlockSpec((8, D), lambda i: (0, 0)),
        out_shape=jax.ShapeDtypeStruct((8, D), _F32),
        scratch_shapes=[pltpu.VMEM((8, D), _F32)],
    )(gate_lo, gate_hi, W_eo, beo)


def _edge_out_body(glo_ref, ghi_ref, w_ref, b_ref, stats_ref, gam_ref,
                   bt_ref, out_ref):
    w = w_ref[...]
    h = (jnp.dot(glo_ref[...], w[:H, :], preferred_element_type=_F32)
         + jnp.dot(ghi_ref[...], w[H:, :], preferred_element_type=_F32)
         + b_ref[...])
    mu = stats_ref[0:1, :] / E
    var = stats_ref[1:2, :] / E - mu * mu
    scale = gam_ref[...] * lax.rsqrt(var + 1e-5)
    out_ref[...] = jnp.maximum((h - mu) * scale + bt_ref[...], 0.0)


def _edge_out(gate_lo, gate_hi, W_eo, beo, stats, ge, bte):
    bn = 4000
    grid = (E // bn,)
    return pl.pallas_call(
        _edge_out_body,
        grid=grid,
        in_specs=[pl.BlockSpec((bn, H), lambda i: (i, 0)),
                  pl.BlockSpec((bn, H), lambda i: (i, 0)),
                  pl.BlockSpec((D, D), lambda i: (0, 0)),
                  pl.BlockSpec((1, D), lambda i: (0, 0)),
                  pl.BlockSpec((8, D), lambda i: (0, 0)),
                  pl.BlockSpec((1, D), lambda i: (0, 0)),
                  pl.BlockSpec((1, D), lambda i: (0, 0))],
        out_specs=pl.BlockSpec((bn, D), lambda i: (i, 0)),
        out_shape=jax.ShapeDtypeStruct((E, D), _F32),
    )(gate_lo, gate_hi, W_eo, beo, stats, ge, bte)


def _node_out_body(xr_ref, gslo_ref, gshi_ref, mslo_ref, mshi_ref,
                   gam_ref, bt_ref, out_ref):
    agg_lo = mslo_ref[...] / (gslo_ref[...] + 1e-6)
    agg_hi = mshi_ref[...] / (gshi_ref[...] + 1e-6)
    t = xr_ref[...] + jnp.concatenate([agg_lo, agg_hi], axis=1)
    mu = jnp.mean(t, axis=0, keepdims=True)
    var = jnp.mean((t - mu) ** 2, axis=0, keepdims=True)
    norm = gam_ref[...] * (t - mu) * lax.rsqrt(var + 1e-5) + bt_ref[...]
    out_ref[...] = jnp.maximum(norm, 0.0)


def _node_out(xr, gs_lo, gs_hi, ms_lo, ms_hi, gn, btn):
    half = pl.BlockSpec((N, H), lambda i: (0, 0))
    return pl.pallas_call(
        _node_out_body,
        grid=(1,),
        in_specs=[pl.BlockSpec((N, D), lambda i: (0, 0)),
                  half, half, half, half,
                  pl.BlockSpec((1, D), lambda i: (0, 0)),
                  pl.BlockSpec((1, D), lambda i: (0, 0))],
        out_specs=pl.BlockSpec((N, D), lambda i: (0, 0)),
        out_shape=jax.ShapeDtypeStruct((N, D), _F32),
    )(xr, gs_lo, gs_hi, ms_lo, ms_hi, gn, btn)


@jax.jit
def kernel(x, edge_index, edge_attr, W_src, b_src, W_dst, b_dst, W_edge,
           b_edge, W_msg, b_msg, W_res, b_res, W_eo, b_eo, gamma_n, beta_n,
           gamma_e, beta_e):
    r = lambda b: b.reshape(1, D)
    tsm_lo, tsm_hi, xd_lo, xd_hi, xr = _node_fwd(
        x, W_src, W_dst, W_msg, W_res, r(b_src), r(b_dst), r(b_msg), r(b_res))
    ea_lo, ea_hi = _edge_fwd(edge_attr, W_edge, r(b_edge))
    gate_lo, gate_hi, gs_lo, gs_hi, ms_lo, ms_hi = _sc_gate(
        edge_index[0], edge_index[1], tsm_lo, tsm_hi, xd_lo, xd_hi,
        ea_lo, ea_hi)
    stats = _edge_stats(gate_lo, gate_hi, W_eo, r(b_eo))
    edge_new = _edge_out(gate_lo, gate_hi, W_eo, r(b_eo), stats,
                         r(gamma_e), r(beta_e))
    x_out = _node_out(xr, gs_lo, gs_hi, ms_lo, ms_hi, r(gamma_n), r(beta_n))
    return (x_out, edge_new)


# speedup vs baseline: 2.4697x; 1.0280x over previous
import jax
import jax.numpy as jnp
from jax import lax
from jax.experimental import pallas as pl
from jax.experimental.pallas import tpu as pltpu
from jax.experimental.pallas import tpu_sc as plsc

N = 10000
E = 320000
D = 128
H = D // 2

NUM_TILES = 16
EPT = E // NUM_TILES
C = 40
NCHUNK = EPT // C
NPAD = 10240
NPT = NPAD // NUM_TILES

_F32 = jnp.float32


def _node_fwd_body(x_ref, ws_ref, wd_ref, wm_ref, wr_ref,
                   bs_ref, bd_ref, bm_ref, br_ref,
                   tsm_lo, tsm_hi, xd_lo, xd_hi, xr_ref):
    xb = x_ref[...]

    def lin(w_ref, b_ref):
        return jnp.dot(xb, w_ref[...], preferred_element_type=_F32) + b_ref[...]

    xs = lin(ws_ref, bs_ref)
    xm = lin(wm_ref, bm_ref)
    tsm_lo[...] = jnp.concatenate([xs[:, :H], xm[:, :H]], axis=1)
    tsm_hi[...] = jnp.concatenate([xs[:, H:], xm[:, H:]], axis=1)
    xd = lin(wd_ref, bd_ref)
    xd_lo[...] = xd[:, :H]
    xd_hi[...] = xd[:, H:]
    xr_ref[...] = lin(wr_ref, br_ref)


def _node_fwd(x, W_src, W_dst, W_msg, W_res, bs, bd, bm, br):
    bn = 2000
    grid = (N // bn,)
    w_spec = pl.BlockSpec((D, D), lambda i: (0, 0))
    b_spec = pl.BlockSpec((1, D), lambda i: (0, 0))
    full = pl.BlockSpec((bn, D), lambda i: (i, 0))
    half = pl.BlockSpec((bn, H), lambda i: (i, 0))
    return pl.pallas_call(
        _node_fwd_body,
        grid=grid,
        in_specs=[full, w_spec, w_spec, w_spec, w_spec,
                  b_spec, b_spec, b_spec, b_spec],
        out_specs=[full, full, half, half, full],
        out_shape=[jax.ShapeDtypeStruct((N, D), _F32)] * 2
        + [jax.ShapeDtypeStruct((N, H), _F32)] * 2
        + [jax.ShapeDtypeStruct((N, D), _F32)],
    )(x, W_src, W_dst, W_msg, W_res, bs, bd, bm, br)


def _edge_fwd_body(a_ref, w_ref, b_ref, lo_ref, hi_ref):
    ea = (jnp.dot(a_ref[...], w_ref[...], preferred_element_type=_F32)
          + b_ref[...])
    lo_ref[...] = ea[:, :H]
    hi_ref[...] = ea[:, H:]


def _edge_fwd(edge_attr, W_edge, be):
    bn = 8000
    grid = (E // bn,)
    return pl.pallas_call(
        _edge_fwd_body,
        grid=grid,
        in_specs=[pl.BlockSpec((bn, D), lambda i: (i, 0)),
                  pl.BlockSpec((D, D), lambda i: (0, 0)),
                  pl.BlockSpec((1, D), lambda i: (0, 0))],
        out_specs=[pl.BlockSpec((bn, H), lambda i: (i, 0)),
                   pl.BlockSpec((bn, H), lambda i: (i, 0))],
        out_shape=[jax.ShapeDtypeStruct((E, H), _F32)] * 2,
    )(edge_attr, W_edge, be)


def _sc_gate_body(src, dst, tsm_lo, tsm_hi, xd_lo, xd_hi, ea_lo, ea_hi,
                  gate_lo, gate_hi, gs_lo, gs_hi, ms_lo, ms_hi,
                  src_idx, dst_idx, sm_g, xd_g, ea_b, gate_b, msg_b,
                  acc_g, acc_m, sem0, sem1, gw0, gw1,
                  isem0, isem1, isem2, isem3, ssem0, ssem1):
    cid = lax.axis_index("c")
    sid = lax.axis_index("s")

    def run_half(tsm_t, xd_t, ea_t, gate_out, gs_out, ms_out):
        sems = (sem0, sem1)
        gws = (gw0, gw1)
        isems = (isem0, isem1, isem2, isem3)
        ssems = (ssem0, ssem1)
        tile_base = sid * EPT

        def zfill(r, _):
            for k in range(H // 16):
                msg_b[0, r, pl.ds(k * 16, 16)] = jnp.zeros((16,), _F32)
            return 0
        lax.fori_loop(0, C, zfill, 0)
        row0 = sid * NPT

        def zcopy(j, _):
            pltpu.sync_copy(msg_b.at[0], acc_g.at[pl.ds(row0 + j * C, C)])
            pltpu.sync_copy(msg_b.at[0], acc_m.at[pl.ds(row0 + j * C, C)])
            return 0
        lax.fori_loop(0, NPT // C, zcopy, 0)
        plsc.subcore_barrier()

        def issue_idx(ib, base):
            pltpu.async_copy(src.at[pl.ds(base, C)], src_idx.at[ib], isems[ib])
            pltpu.async_copy(dst.at[pl.ds(base, C)], dst_idx.at[ib], isems[ib])

        def issue_gather(b, ib, base):
            pltpu.make_async_copy(
                src.at[pl.ds(0, C)], src_idx.at[ib], isems[ib]).wait()
            pltpu.make_async_copy(
                dst.at[pl.ds(0, C)], dst_idx.at[ib], isems[ib]).wait()
            pltpu.async_copy(tsm_t.at[src_idx.at[ib]], sm_g.at[b], sems[b])
            pltpu.async_copy(xd_t.at[dst_idx.at[ib]], xd_g.at[b], sems[b])
            pltpu.async_copy(ea_t.at[pl.ds(base, C)], ea_b.at[b], sems[b])

        def wait_in(b, ib):
            pltpu.make_async_copy(
                tsm_t.at[src_idx.at[ib]], sm_g.at[b], sems[b]).wait()
            pltpu.make_async_copy(
                xd_t.at[dst_idx.at[ib]], xd_g.at[b], sems[b]).wait()
            pltpu.make_async_copy(
                ea_t.at[pl.ds(0, C)], ea_b.at[b], sems[b]).wait()

        def wait_scatters(b, ib):
            pltpu.make_async_copy(
                gate_b.at[b], acc_g.at[dst_idx.at[ib]], ssems[b]).wait()
            pltpu.make_async_copy(
                msg_b.at[b], acc_m.at[dst_idx.at[ib]], ssems[b]).wait()

        def do_chunk(b, ib, i, base):
            wait_in(b, ib)

            @pl.when(i >= 2)
            def _():
                pltpu.make_async_copy(
                    gate_b.at[b], gate_out.at[pl.ds(0, C)], gws[b]).wait()
                wait_scatters(b, ib)

            def comp(rr, _):
                rk = [(r, k) for r in (2 * rr, 2 * rr + 1)
                      for k in range(H // 16)]
                ts = [sm_g[b, r, pl.ds(k * 16, 16)]
                      + xd_g[b, r, pl.ds(k * 16, 16)]
                      + ea_b[b, r, pl.ds(k * 16, 16)] for r, k in rk]
                es = [jnp.exp(-t) for t in ts]
                gs = [1.0 / (1.0 + e) for e in es]
                ms = [sm_g[b, r, pl.ds(H + k * 16, 16)] * g
                      for (r, k), g in zip(rk, gs)]
                for (r, k), g, m in zip(rk, gs, ms):
                    gate_b[b, r, pl.ds(k * 16, 16)] = g
                    msg_b[b, r, pl.ds(k * 16, 16)] = m
                return 0
            lax.fori_loop(0, C // 2, comp, 0)

            pltpu.async_copy(gate_b.at[b], acc_g.at[dst_idx.at[ib]],
                             ssems[b], add=True)
            pltpu.async_copy(msg_b.at[b], acc_m.at[dst_idx.at[ib]],
                             ssems[b], add=True)
            pltpu.async_copy(gate_b.at[b], gate_out.at[pl.ds(base, C)], gws[b])

        issue_idx(0, tile_base)
        issue_idx(1, tile_base + C)
        issue_gather(0, 0, tile_base)

        def step(i, _):
            m = lax.rem(i, 4)
            for k in range(4):
                @pl.when(m == k)
                def _(k=k):
                    issue_gather((k + 1) % 2, (k + 1) % 4,
                                 tile_base + lax.rem(i + 1, NCHUNK) * C)
                    do_chunk(k % 2, k, i, tile_base + i * C)
                    issue_idx((k + 2) % 4,
                              tile_base + lax.rem(i + 2, NCHUNK) * C)
            return 0
        lax.fori_loop(0, NCHUNK, step, 0)

        wait_in(0, 0)
        pltpu.make_async_copy(
            src.at[pl.ds(0, C)], src_idx.at[1], isems[1]).wait()
        pltpu.make_async_copy(
            dst.at[pl.ds(0, C)], dst_idx.at[1], isems[1]).wait()
        pltpu.make_async_copy(
            gate_b.at[0], gate_out.at[pl.ds(0, C)], gws[0]).wait()
        pltpu.make_async_copy(
            gate_b.at[1], gate_out.at[pl.ds(0, C)], gws[1]).wait()
        wait_scatters(0, 2)
        wait_scatters(1, 3)

        plsc.subcore_barrier()
        pltpu.sync_copy(acc_g.at[pl.ds(row0, NPT)], gs_out.at[pl.ds(row0, NPT)])
        pltpu.sync_copy(acc_m.at[pl.ds(row0, NPT)], ms_out.at[pl.ds(row0, NPT)])

    @pl.when(cid == 0)
    def _():
        run_half(tsm_lo, xd_lo, ea_lo, gate_lo, gs_lo, ms_lo)

    @pl.when(cid == 1)
    def _():
        run_half(tsm_hi, xd_hi, ea_hi, gate_hi, gs_hi, ms_hi)


def _sc_gate(src, dst, tsm_lo, tsm_hi, xd_lo, xd_hi, ea_lo, ea_hi):
    mesh = plsc.VectorSubcoreMesh(core_axis_name="c", subcore_axis_name="s")
    f = pl.kernel(
        _sc_gate_body,
        out_type=[jax.ShapeDtypeStruct((E, H), _F32)] * 2
        + [jax.ShapeDtypeStruct((NPAD, H), _F32)] * 4,
        mesh=mesh,
        scratch_types=[
            pltpu.VMEM((4, C), jnp.int32),
            pltpu.VMEM((4, C), jnp.int32),
            pltpu.VMEM((2, C, D), _F32),
            pltpu.VMEM((2, C, H), _F32),
            pltpu.VMEM((2, C, H), _F32),
            pltpu.VMEM((2, C, H), _F32),
            pltpu.VMEM((2, C, H), _F32),
            pltpu.VMEM_SHARED((NPAD, H), _F32),
            pltpu.VMEM_SHARED((NPAD, H), _F32),
            pltpu.SemaphoreType.DMA,
            pltpu.SemaphoreType.DMA,
            pltpu.SemaphoreType.DMA,
            pltpu.SemaphoreType.DMA,
            pltpu.SemaphoreType.DMA,
            pltpu.SemaphoreType.DMA,
            pltpu.SemaphoreType.DMA,
            pltpu.SemaphoreType.DMA,
            pltpu.SemaphoreType.DMA,
            pltpu.SemaphoreType.DMA,
        ],
        compiler_params=pltpu.CompilerParams(use_tc_tiling_on_sc=False),
    )
    return f(src, dst, tsm_lo, tsm_hi, xd_lo, xd_hi, ea_lo, ea_hi)


def _edge_stats_body(glo_ref, ghi_ref, w_ref, b_ref, stats_ref, acc_ref):
    i = pl.program_id(0)

    @pl.when(i == 0)
    def _():
        acc_ref[...] = jnp.zeros_like(acc_ref)

    w = w_ref[...]
    h = (jnp.dot(glo_ref[...], w[:H, :], preferred_element_type=_F32)
         + jnp.dot(ghi_ref[...], w[H:, :], preferred_element_type=_F32)
         + b_ref[...])
    acc_ref[0:1, :] += jnp.sum(h, axis=0, keepdims=True)
    acc_ref[1:2, :] += jnp.sum(h * h, axis=0, keepdims=True)

    @pl.when(i == pl.num_programs(0) - 1)
    def _():
        stats_ref[...] = acc_ref[...]


def _edge_stats(gate_lo, gate_hi, W_eo, beo):
    bn = 8000
    grid = (E // bn,)
    return pl.pallas_call(
        _edge_stats_body,
        grid=grid,
        in_specs=[pl.BlockSpec((bn, H), lambda i: (i, 0)),
                  pl.BlockSpec((bn, H), lambda i: (i, 0)),
                  pl.BlockSpec((D, D), lambda i: (0, 0)),
                  pl.BlockSpec((1, D), lambda i: (0, 0))],
        out_specs=pl.BlockSpec((8, D), lambda i: (0, 0)),
        out_shape=jax.ShapeDtypeStruct((8, D), _F32),
        scratch_shapes=[pltpu.VMEM((8, D), _F32)],
    )(gate_lo, gate_hi, W_eo, beo)


def _edge_out_body(glo_ref, ghi_ref, w_ref, b_ref, stats_ref, gam_ref,
                   bt_ref, out_ref):
    w = w_ref[...]
    h = (jnp.dot(glo_ref[...], w[:H, :], preferred_element_type=_F32)
         + jnp.dot(ghi_ref[...], w[H:, :], preferred_element_type=_F32)
         + b_ref[...])
    mu = stats_ref[0:1, :] / E
    var = stats_ref[1:2, :] / E - mu * mu
    scale = gam_ref[...] * lax.rsqrt(var + 1e-5)
    out_ref[...] = jnp.maximum((h - mu) * scale + bt_ref[...], 0.0)


def _edge_out(gate_lo, gate_hi, W_eo, beo, stats, ge, bte):
    bn = 8000
    grid = (E // bn,)
    return pl.pallas_call(
        _edge_out_body,
        grid=grid,
        in_specs=[pl.BlockSpec((bn, H), lambda i: (i, 0)),
                  pl.BlockSpec((bn, H), lambda i: (i, 0)),
                  pl.BlockSpec((D, D), lambda i: (0, 0)),
                  pl.BlockSpec((1, D), lambda i: (0, 0)),
                  pl.BlockSpec((8, D), lambda i: (0, 0)),
                  pl.BlockSpec((1, D), lambda i: (0, 0)),
                  pl.BlockSpec((1, D), lambda i: (0, 0))],
        out_specs=pl.BlockSpec((bn, D), lambda i: (i, 0)),
        out_shape=jax.ShapeDtypeStruct((E, D), _F32),
    )(gate_lo, gate_hi, W_eo, beo, stats, ge, bte)


def _node_out_body(xr_ref, gslo_ref, gshi_ref, mslo_ref, mshi_ref,
                   gam_ref, bt_ref, out_ref):
    agg_lo = mslo_ref[...] / (gslo_ref[...] + 1e-6)
    agg_hi = mshi_ref[...] / (gshi_ref[...] + 1e-6)
    t = xr_ref[...] + jnp.concatenate([agg_lo, agg_hi], axis=1)
    mu = jnp.mean(t, axis=0, keepdims=True)
    var = jnp.mean((t - mu) ** 2, axis=0, keepdims=True)
    norm = gam_ref[...] * (t - mu) * lax.rsqrt(var + 1e-5) + bt_ref[...]
    out_ref[...] = jnp.maximum(norm, 0.0)


def _node_out(xr, gs_lo, gs_hi, ms_lo, ms_hi, gn, btn):
    half = pl.BlockSpec((N, H), lambda i: (0, 0))
    return pl.pallas_call(
        _node_out_body,
        grid=(1,),
        in_specs=[pl.BlockSpec((N, D), lambda i: (0, 0)),
                  half, half, half, half,
                  pl.BlockSpec((1, D), lambda i: (0, 0)),
                  pl.BlockSpec((1, D), lambda i: (0, 0))],
        out_specs=pl.BlockSpec((N, D), lambda i: (0, 0)),
        out_shape=jax.ShapeDtypeStruct((N, D), _F32),
    )(xr, gs_lo, gs_hi, ms_lo, ms_hi, gn, btn)


@jax.jit
def kernel(x, edge_index, edge_attr, W_src, b_src, W_dst, b_dst, W_edge,
           b_edge, W_msg, b_msg, W_res, b_res, W_eo, b_eo, gamma_n, beta_n,
           gamma_e, beta_e):
    r = lambda b: b.reshape(1, D)
    tsm_lo, tsm_hi, xd_lo, xd_hi, xr = _node_fwd(
        x, W_src, W_dst, W_msg, W_res, r(b_src), r(b_dst), r(b_msg), r(b_res))
    ea_lo, ea_hi = _edge_fwd(edge_attr, W_edge, r(b_edge))
    gate_lo, gate_hi, gs_lo, gs_hi, ms_lo, ms_hi = _sc_gate(
        edge_index[0], edge_index[1], tsm_lo, tsm_hi, xd_lo, xd_hi,
        ea_lo, ea_hi)
    stats = _edge_stats(gate_lo, gate_hi, W_eo, r(b_eo))
    edge_new = _edge_out(gate_lo, gate_hi, W_eo, r(b_eo), stats,
                         r(gamma_e), r(beta_e))
    x_out = _node_out(xr, gs_lo, gs_hi, ms_lo, ms_hi, r(gamma_n), r(beta_n))
    return (x_out, edge_new)


# speedup vs baseline: 2.5423x; 1.0294x over previous
import jax
import jax.numpy as jnp
from jax import lax
from jax.experimental import pallas as pl
from jax.experimental.pallas import tpu as pltpu
from jax.experimental.pallas import tpu_sc as plsc

N = 10000
E = 320000
D = 128
H = D // 2

NUM_TILES = 16
EPT = E // NUM_TILES
C = 40
NCHUNK = EPT // C
NPAD = 10240
NPT = NPAD // NUM_TILES

_F32 = jnp.float32


def _node_fwd_body(x_ref, ws_ref, wd_ref, wm_ref, wr_ref,
                   bs_ref, bd_ref, bm_ref, br_ref,
                   tsm_lo, tsm_hi, xd_lo, xd_hi, xr_ref):
    xb = x_ref[...]

    def lin(w_ref, b_ref):
        return jnp.dot(xb, w_ref[...], preferred_element_type=_F32) + b_ref[...]

    xs = lin(ws_ref, bs_ref)
    xm = lin(wm_ref, bm_ref)
    tsm_lo[...] = jnp.concatenate([xs[:, :H], xm[:, :H]], axis=1)
    tsm_hi[...] = jnp.concatenate([xs[:, H:], xm[:, H:]], axis=1)
    xd = lin(wd_ref, bd_ref)
    xd_lo[...] = xd[:, :H]
    xd_hi[...] = xd[:, H:]
    xr_ref[...] = lin(wr_ref, br_ref)


def _node_fwd(x, W_src, W_dst, W_msg, W_res, bs, bd, bm, br):
    bn = 2000
    grid = (N // bn,)
    w_spec = pl.BlockSpec((D, D), lambda i: (0, 0))
    b_spec = pl.BlockSpec((1, D), lambda i: (0, 0))
    full = pl.BlockSpec((bn, D), lambda i: (i, 0))
    half = pl.BlockSpec((bn, H), lambda i: (i, 0))
    return pl.pallas_call(
        _node_fwd_body,
        grid=grid,
        in_specs=[full, w_spec, w_spec, w_spec, w_spec,
                  b_spec, b_spec, b_spec, b_spec],
        out_specs=[full, full, half, half, full],
        out_shape=[jax.ShapeDtypeStruct((N, D), _F32)] * 2
        + [jax.ShapeDtypeStruct((N, H), _F32)] * 2
        + [jax.ShapeDtypeStruct((N, D), _F32)],
    )(x, W_src, W_dst, W_msg, W_res, bs, bd, bm, br)


def _edge_fwd_body(a_ref, w_ref, b_ref, lo_ref, hi_ref):
    ea = (jnp.dot(a_ref[...], w_ref[...], preferred_element_type=_F32)
          + b_ref[...])
    lo_ref[...] = ea[:, :H]
    hi_ref[...] = ea[:, H:]


def _edge_fwd(edge_attr, W_edge, be):
    bn = 8000
    grid = (E // bn,)
    return pl.pallas_call(
        _edge_fwd_body,
        grid=grid,
        in_specs=[pl.BlockSpec((bn, D), lambda i: (i, 0)),
                  pl.BlockSpec((D, D), lambda i: (0, 0)),
                  pl.BlockSpec((1, D), lambda i: (0, 0))],
        out_specs=[pl.BlockSpec((bn, H), lambda i: (i, 0)),
                   pl.BlockSpec((bn, H), lambda i: (i, 0))],
        out_shape=[jax.ShapeDtypeStruct((E, H), _F32)] * 2,
    )(edge_attr, W_edge, be)


def _sc_gate_body(src, dst, tsm_lo, tsm_hi, xd_lo, xd_hi, ea_lo, ea_hi,
                  gate_lo, gate_hi, gs_lo, gs_hi, ms_lo, ms_hi,
                  src_idx, dst_idx, sm_g, xd_g, ea_b, gate_b, msg_b,
                  acc_g, acc_m, sem0, sem1, gw0, gw1,
                  isem0, isem1, isem2, isem3, ssem0, ssem1):
    cid = lax.axis_index("c")
    sid = lax.axis_index("s")

    def run_half(tsm_t, xd_t, ea_t, gate_out, gs_out, ms_out):
        sems = (sem0, sem1)
        gws = (gw0, gw1)
        isems = (isem0, isem1, isem2, isem3)
        ssems = (ssem0, ssem1)
        tile_base = sid * EPT

        def zfill(r, _):
            for k in range(H // 16):
                msg_b[0, r, pl.ds(k * 16, 16)] = jnp.zeros((16,), _F32)
            return 0
        lax.fori_loop(0, C, zfill, 0)
        row0 = sid * NPT

        def zcopy(j, _):
            pltpu.sync_copy(msg_b.at[0], acc_g.at[pl.ds(row0 + j * C, C)])
            pltpu.sync_copy(msg_b.at[0], acc_m.at[pl.ds(row0 + j * C, C)])
            return 0
        lax.fori_loop(0, NPT // C, zcopy, 0)
        plsc.subcore_barrier()

        def issue_idx(ib, base):
            pltpu.async_copy(src.at[pl.ds(base, C)], src_idx.at[ib], isems[ib])
            pltpu.async_copy(dst.at[pl.ds(base, C)], dst_idx.at[ib], isems[ib])

        def issue_gather(b, ib, base):
            pltpu.make_async_copy(
                src.at[pl.ds(0, C)], src_idx.at[ib], isems[ib]).wait()
            pltpu.make_async_copy(
                dst.at[pl.ds(0, C)], dst_idx.at[ib], isems[ib]).wait()
            pltpu.async_copy(tsm_t.at[src_idx.at[ib]], sm_g.at[b], sems[b])
            pltpu.async_copy(xd_t.at[dst_idx.at[ib]], xd_g.at[b], sems[b])
            pltpu.async_copy(ea_t.at[pl.ds(base, C)], ea_b.at[b], sems[b])

        def wait_in(b, ib):
            pltpu.make_async_copy(
                tsm_t.at[src_idx.at[ib]], sm_g.at[b], sems[b]).wait()
            pltpu.make_async_copy(
                xd_t.at[dst_idx.at[ib]], xd_g.at[b], sems[b]).wait()
            pltpu.make_async_copy(
                ea_t.at[pl.ds(0, C)], ea_b.at[b], sems[b]).wait()

        def wait_scatters(b, ib):
            pltpu.make_async_copy(
                gate_b.at[b], acc_g.at[dst_idx.at[ib]], ssems[b]).wait()
            pltpu.make_async_copy(
                msg_b.at[b], acc_m.at[dst_idx.at[ib]], ssems[b]).wait()

        def do_chunk(b, ib, i, base):
            wait_in(b, ib)

            @pl.when(i >= 2)
            def _():
                pltpu.make_async_copy(
                    gate_b.at[b], gate_out.at[pl.ds(0, C)], gws[b]).wait()
                wait_scatters(b, ib)

            def comp(rr, _):
                rk = [(r, k) for r in (2 * rr, 2 * rr + 1)
                      for k in range(H // 16)]
                ts = [sm_g[b, r, pl.ds(k * 16, 16)]
                      + xd_g[b, r, pl.ds(k * 16, 16)]
                      + ea_b[b, r, pl.ds(k * 16, 16)] for r, k in rk]
                es = [jnp.exp(-t) for t in ts]
                gs = [1.0 / (1.0 + e) for e in es]
                ms = [sm_g[b, r, pl.ds(H + k * 16, 16)] * g
                      for (r, k), g in zip(rk, gs)]
                for (r, k), g, m in zip(rk, gs, ms):
                    gate_b[b, r, pl.ds(k * 16, 16)] = g
                    msg_b[b, r, pl.ds(k * 16, 16)] = m
                return 0
            lax.fori_loop(0, C // 2, comp, 0)

            pltpu.async_copy(gate_b.at[b], acc_g.at[dst_idx.at[ib]],
                             ssems[b], add=True)
            pltpu.async_copy(msg_b.at[b], acc_m.at[dst_idx.at[ib]],
                             ssems[b], add=True)
            pltpu.async_copy(gate_b.at[b], gate_out.at[pl.ds(base, C)], gws[b])

        issue_idx(0, tile_base)
        issue_idx(1, tile_base + C)
        issue_gather(0, 0, tile_base)

        def step(i, _):
            m = lax.rem(i, 4)
            for k in range(4):
                @pl.when(m == k)
                def _(k=k):
                    issue_gather((k + 1) % 2, (k + 1) % 4,
                                 tile_base + lax.rem(i + 1, NCHUNK) * C)
                    do_chunk(k % 2, k, i, tile_base + i * C)
                    issue_idx((k + 2) % 4,
                              tile_base + lax.rem(i + 2, NCHUNK) * C)
            return 0
        lax.fori_loop(0, NCHUNK, step, 0)

        wait_in(0, 0)
        pltpu.make_async_copy(
            src.at[pl.ds(0, C)], src_idx.at[1], isems[1]).wait()
        pltpu.make_async_copy(
            dst.at[pl.ds(0, C)], dst_idx.at[1], isems[1]).wait()
        pltpu.make_async_copy(
            gate_b.at[0], gate_out.at[pl.ds(0, C)], gws[0]).wait()
        pltpu.make_async_copy(
            gate_b.at[1], gate_out.at[pl.ds(0, C)], gws[1]).wait()
        wait_scatters(0, 2)
        wait_scatters(1, 3)

        plsc.subcore_barrier()
        pltpu.sync_copy(acc_g.at[pl.ds(row0, NPT)], gs_out.at[pl.ds(row0, NPT)])
        pltpu.sync_copy(acc_m.at[pl.ds(row0, NPT)], ms_out.at[pl.ds(row0, NPT)])

    @pl.when(cid == 0)
    def _():
        run_half(tsm_lo, xd_lo, ea_lo, gate_lo, gs_lo, ms_lo)

    @pl.when(cid == 1)
    def _():
        run_half(tsm_hi, xd_hi, ea_hi, gate_hi, gs_hi, ms_hi)


def _sc_gate(src, dst, tsm_lo, tsm_hi, xd_lo, xd_hi, ea_lo, ea_hi):
    mesh = plsc.VectorSubcoreMesh(core_axis_name="c", subcore_axis_name="s")
    f = pl.kernel(
        _sc_gate_body,
        out_type=[jax.ShapeDtypeStruct((E, H), _F32)] * 2
        + [jax.ShapeDtypeStruct((NPAD, H), _F32)] * 4,
        mesh=mesh,
        scratch_types=[
            pltpu.VMEM((4, C), jnp.int32),
            pltpu.VMEM((4, C), jnp.int32),
            pltpu.VMEM((2, C, D), _F32),
            pltpu.VMEM((2, C, H), _F32),
            pltpu.VMEM((2, C, H), _F32),
            pltpu.VMEM((2, C, H), _F32),
            pltpu.VMEM((2, C, H), _F32),
            pltpu.VMEM_SHARED((NPAD, H), _F32),
            pltpu.VMEM_SHARED((NPAD, H), _F32),
            pltpu.SemaphoreType.DMA,
            pltpu.SemaphoreType.DMA,
            pltpu.SemaphoreType.DMA,
            pltpu.SemaphoreType.DMA,
            pltpu.SemaphoreType.DMA,
            pltpu.SemaphoreType.DMA,
            pltpu.SemaphoreType.DMA,
            pltpu.SemaphoreType.DMA,
            pltpu.SemaphoreType.DMA,
            pltpu.SemaphoreType.DMA,
        ],
        compiler_params=pltpu.CompilerParams(use_tc_tiling_on_sc=False),
    )
    return f(src, dst, tsm_lo, tsm_hi, xd_lo, xd_hi, ea_lo, ea_hi)


def _edge_stats_body(glo_ref, ghi_ref, w_ref, b_ref, stats_ref, h16_ref,
                     acc_ref):
    i = pl.program_id(0)

    @pl.when(i == 0)
    def _():
        acc_ref[...] = jnp.zeros_like(acc_ref)

    w = w_ref[...]
    h = (jnp.dot(glo_ref[...], w[:H, :], preferred_element_type=_F32)
         + jnp.dot(ghi_ref[...], w[H:, :], preferred_element_type=_F32)
         + b_ref[...])
    h16_ref[...] = h.astype(jnp.bfloat16)
    acc_ref[0:1, :] += jnp.sum(h, axis=0, keepdims=True)
    acc_ref[1:2, :] += jnp.sum(h * h, axis=0, keepdims=True)

    @pl.when(i == pl.num_programs(0) - 1)
    def _():
        stats_ref[...] = acc_ref[...]


def _edge_stats(gate_lo, gate_hi, W_eo, beo):
    bn = 8000
    grid = (E // bn,)
    return pl.pallas_call(
        _edge_stats_body,
        grid=grid,
        in_specs=[pl.BlockSpec((bn, H), lambda i: (i, 0)),
                  pl.BlockSpec((bn, H), lambda i: (i, 0)),
                  pl.BlockSpec((D, D), lambda i: (0, 0)),
                  pl.BlockSpec((1, D), lambda i: (0, 0))],
        out_specs=[pl.BlockSpec((8, D), lambda i: (0, 0)),
                   pl.BlockSpec((bn, D), lambda i: (i, 0))],
        out_shape=[jax.ShapeDtypeStruct((8, D), _F32),
                   jax.ShapeDtypeStruct((E, D), jnp.bfloat16)],
        scratch_shapes=[pltpu.VMEM((8, D), _F32)],
    )(gate_lo, gate_hi, W_eo, beo)


def _edge_out_body(h16_ref, stats_ref, gam_ref, bt_ref, out_ref):
    h = h16_ref[...].astype(_F32)
    mu = stats_ref[0:1, :] / E
    var = stats_ref[1:2, :] / E - mu * mu
    scale = gam_ref[...] * lax.rsqrt(var + 1e-5)
    out_ref[...] = jnp.maximum((h - mu) * scale + bt_ref[...], 0.0)


def _edge_out(h16, stats, ge, bte):
    bn = 8000
    grid = (E // bn,)
    return pl.pallas_call(
        _edge_out_body,
        grid=grid,
        in_specs=[pl.BlockSpec((bn, D), lambda i: (i, 0)),
                  pl.BlockSpec((8, D), lambda i: (0, 0)),
                  pl.BlockSpec((1, D), lambda i: (0, 0)),
                  pl.BlockSpec((1, D), lambda i: (0, 0))],
        out_specs=pl.BlockSpec((bn, D), lambda i: (i, 0)),
        out_shape=jax.ShapeDtypeStruct((E, D), _F32),
    )(h16, stats, ge, bte)


def _node_out_body(xr_ref, gslo_ref, gshi_ref, mslo_ref, mshi_ref,
                   gam_ref, bt_ref, out_ref):
    agg_lo = mslo_ref[...] / (gslo_ref[...] + 1e-6)
    agg_hi = mshi_ref[...] / (gshi_ref[...] + 1e-6)
    t = xr_ref[...] + jnp.concatenate([agg_lo, agg_hi], axis=1)
    mu = jnp.mean(t, axis=0, keepdims=True)
    var = jnp.mean((t - mu) ** 2, axis=0, keepdims=True)
    norm = gam_ref[...] * (t - mu) * lax.rsqrt(var + 1e-5) + bt_ref[...]
    out_ref[...] = jnp.maximum(norm, 0.0)


def _node_out(xr, gs_lo, gs_hi, ms_lo, ms_hi, gn, btn):
    half = pl.BlockSpec((N, H), lambda i: (0, 0))
    return pl.pallas_call(
        _node_out_body,
        grid=(1,),
        in_specs=[pl.BlockSpec((N, D), lambda i: (0, 0)),
                  half, half, half, half,
                  pl.BlockSpec((1, D), lambda i: (0, 0)),
                  pl.BlockSpec((1, D), lambda i: (0, 0))],
        out_specs=pl.BlockSpec((N, D), lambda i: (0, 0)),
        out_shape=jax.ShapeDtypeStruct((N, D), _F32),
    )(xr, gs_lo, gs_hi, ms_lo, ms_hi, gn, btn)


@jax.jit
def kernel(x, edge_index, edge_attr, W_src, b_src, W_dst, b_dst, W_edge,
           b_edge, W_msg, b_msg, W_res, b_res, W_eo, b_eo, gamma_n, beta_n,
           gamma_e, beta_e):
    r = lambda b: b.reshape(1, D)
    tsm_lo, tsm_hi, xd_lo, xd_hi, xr = _node_fwd(
        x, W_src, W_dst, W_msg, W_res, r(b_src), r(b_dst), r(b_msg), r(b_res))
    ea_lo, ea_hi = _edge_fwd(edge_attr, W_edge, r(b_edge))
    gate_lo, gate_hi, gs_lo, gs_hi, ms_lo, ms_hi = _sc_gate(
        edge_index[0], edge_index[1], tsm_lo, tsm_hi, xd_lo, xd_hi,
        ea_lo, ea_hi)
    stats, h16 = _edge_stats(gate_lo, gate_hi, W_eo, r(b_eo))
    edge_new = _edge_out(h16, stats, r(gamma_e), r(beta_e))
    x_out = _node_out(xr, gs_lo, gs_hi, ms_lo, ms_hi, r(gamma_n), r(beta_n))
    return (x_out, edge_new)


# speedup vs baseline: 2.5582x; 1.0063x over previous
import jax
import jax.numpy as jnp
from jax import lax
from jax.experimental import pallas as pl
from jax.experimental.pallas import tpu as pltpu
from jax.experimental.pallas import tpu_sc as plsc

N = 10000
E = 320000
D = 128
H = D // 2

NUM_TILES = 16
EPT = E // NUM_TILES
C = 40
NCHUNK = EPT // C
NPAD = 10240
NPT = NPAD // NUM_TILES

_F32 = jnp.float32


def _node_fwd_body(x_ref, ws_ref, wd_ref, wm_ref, wr_ref,
                   bs_ref, bd_ref, bm_ref, br_ref,
                   tsm_lo, tsm_hi, xd_lo, xd_hi, xr_ref):
    xb = x_ref[...]

    def lin(w_ref, b_ref):
        return jnp.dot(xb, w_ref[...], preferred_element_type=_F32) + b_ref[...]

    xs = lin(ws_ref, bs_ref)
    xm = lin(wm_ref, bm_ref)
    tsm_lo[...] = jnp.concatenate([xs[:, :H], xm[:, :H]], axis=1)
    tsm_hi[...] = jnp.concatenate([xs[:, H:], xm[:, H:]], axis=1)
    xd = lin(wd_ref, bd_ref)
    xd_lo[...] = xd[:, :H]
    xd_hi[...] = xd[:, H:]
    xr_ref[...] = lin(wr_ref, br_ref)


def _node_fwd(x, W_src, W_dst, W_msg, W_res, bs, bd, bm, br):
    bn = 2000
    grid = (N // bn,)
    w_spec = pl.BlockSpec((D, D), lambda i: (0, 0))
    b_spec = pl.BlockSpec((1, D), lambda i: (0, 0))
    full = pl.BlockSpec((bn, D), lambda i: (i, 0))
    half = pl.BlockSpec((bn, H), lambda i: (i, 0))
    return pl.pallas_call(
        _node_fwd_body,
        grid=grid,
        in_specs=[full, w_spec, w_spec, w_spec, w_spec,
                  b_spec, b_spec, b_spec, b_spec],
        out_specs=[full, full, half, half, full],
        out_shape=[jax.ShapeDtypeStruct((N, D), _F32)] * 2
        + [jax.ShapeDtypeStruct((N, H), _F32)] * 2
        + [jax.ShapeDtypeStruct((N, D), _F32)],
    )(x, W_src, W_dst, W_msg, W_res, bs, bd, bm, br)


def _edge_fwd_body(a_ref, w_ref, b_ref, lo_ref, hi_ref):
    ea = (jnp.dot(a_ref[...], w_ref[...], preferred_element_type=_F32)
          + b_ref[...])
    lo_ref[...] = ea[:, :H]
    hi_ref[...] = ea[:, H:]


def _edge_fwd(edge_attr, W_edge, be):
    bn = 16000
    grid = (E // bn,)
    return pl.pallas_call(
        _edge_fwd_body,
        grid=grid,
        in_specs=[pl.BlockSpec((bn, D), lambda i: (i, 0)),
                  pl.BlockSpec((D, D), lambda i: (0, 0)),
                  pl.BlockSpec((1, D), lambda i: (0, 0))],
        out_specs=[pl.BlockSpec((bn, H), lambda i: (i, 0)),
                   pl.BlockSpec((bn, H), lambda i: (i, 0))],
        out_shape=[jax.ShapeDtypeStruct((E, H), _F32)] * 2,
    )(edge_attr, W_edge, be)


def _sc_gate_body(src, dst, tsm_lo, tsm_hi, xd_lo, xd_hi, ea_lo, ea_hi,
                  gate_lo, gate_hi, gs_lo, gs_hi, ms_lo, ms_hi,
                  src_idx, dst_idx, sm_g, xd_g, ea_b, gate_b, msg_b,
                  acc_g, acc_m, sem0, sem1, gw0, gw1,
                  isem0, isem1, isem2, isem3, ssem0, ssem1):
    cid = lax.axis_index("c")
    sid = lax.axis_index("s")

    def run_half(tsm_t, xd_t, ea_t, gate_out, gs_out, ms_out):
        sems = (sem0, sem1)
        gws = (gw0, gw1)
        isems = (isem0, isem1, isem2, isem3)
        ssems = (ssem0, ssem1)
        tile_base = sid * EPT

        def zfill(r, _):
            for k in range(H // 16):
                msg_b[0, r, pl.ds(k * 16, 16)] = jnp.zeros((16,), _F32)
            return 0
        lax.fori_loop(0, C, zfill, 0)
        row0 = sid * NPT

        def zcopy(j, _):
            pltpu.sync_copy(msg_b.at[0], acc_g.at[pl.ds(row0 + j * C, C)])
            pltpu.sync_copy(msg_b.at[0], acc_m.at[pl.ds(row0 + j * C, C)])
            return 0
        lax.fori_loop(0, NPT // C, zcopy, 0)
        plsc.subcore_barrier()

        def issue_idx(ib, base):
            pltpu.async_copy(src.at[pl.ds(base, C)], src_idx.at[ib], isems[ib])
            pltpu.async_copy(dst.at[pl.ds(base, C)], dst_idx.at[ib], isems[ib])

        def issue_gather(b, ib, base):
            pltpu.make_async_copy(
                src.at[pl.ds(0, C)], src_idx.at[ib], isems[ib]).wait()
            pltpu.make_async_copy(
                dst.at[pl.ds(0, C)], dst_idx.at[ib], isems[ib]).wait()
            pltpu.async_copy(tsm_t.at[src_idx.at[ib]], sm_g.at[b], sems[b])
            pltpu.async_copy(xd_t.at[dst_idx.at[ib]], xd_g.at[b], sems[b])
            pltpu.async_copy(ea_t.at[pl.ds(base, C)], ea_b.at[b], sems[b])

        def wait_in(b, ib):
            pltpu.make_async_copy(
                tsm_t.at[src_idx.at[ib]], sm_g.at[b], sems[b]).wait()
            pltpu.make_async_copy(
                xd_t.at[dst_idx.at[ib]], xd_g.at[b], sems[b]).wait()
            pltpu.make_async_copy(
                ea_t.at[pl.ds(0, C)], ea_b.at[b], sems[b]).wait()

        def wait_scatters(b, ib):
            pltpu.make_async_copy(
                gate_b.at[b], acc_g.at[dst_idx.at[ib]], ssems[b]).wait()
            pltpu.make_async_copy(
                msg_b.at[b], acc_m.at[dst_idx.at[ib]], ssems[b]).wait()

        def do_chunk(b, ib, i, base):
            wait_in(b, ib)

            @pl.when(i >= 2)
            def _():
                pltpu.make_async_copy(
                    gate_b.at[b], gate_out.at[pl.ds(0, C)], gws[b]).wait()
                wait_scatters(b, ib)

            def comp(rr, _):
                rk = [(r, k) for r in (2 * rr, 2 * rr + 1)
                      for k in range(H // 16)]
                ts = [sm_g[b, r, pl.ds(k * 16, 16)]
                      + xd_g[b, r, pl.ds(k * 16, 16)]
                      + ea_b[b, r, pl.ds(k * 16, 16)] for r, k in rk]
                es = [jnp.exp(-t) for t in ts]
                gs = [1.0 / (1.0 + e) for e in es]
                ms = [sm_g[b, r, pl.ds(H + k * 16, 16)] * g
                      for (r, k), g in zip(rk, gs)]
                for (r, k), g, m in zip(rk, gs, ms):
                    gate_b[b, r, pl.ds(k * 16, 16)] = g
                    msg_b[b, r, pl.ds(k * 16, 16)] = m
                return 0
            lax.fori_loop(0, C // 2, comp, 0)

            pltpu.async_copy(gate_b.at[b], acc_g.at[dst_idx.at[ib]],
                             ssems[b], add=True)
            pltpu.async_copy(msg_b.at[b], acc_m.at[dst_idx.at[ib]],
                             ssems[b], add=True)
            pltpu.async_copy(gate_b.at[b], gate_out.at[pl.ds(base, C)], gws[b])

        issue_idx(0, tile_base)
        issue_idx(1, tile_base + C)
        issue_gather(0, 0, tile_base)

        def step(i, _):
            m = lax.rem(i, 4)
            for k in range(4):
                @pl.when(m == k)
                def _(k=k):
                    issue_gather((k + 1) % 2, (k + 1) % 4,
                                 tile_base + lax.rem(i + 1, NCHUNK) * C)
                    do_chunk(k % 2, k, i, tile_base + i * C)
                    issue_idx((k + 2) % 4,
                              tile_base + lax.rem(i + 2, NCHUNK) * C)
            return 0
        lax.fori_loop(0, NCHUNK, step, 0)

        wait_in(0, 0)
        pltpu.make_async_copy(
            src.at[pl.ds(0, C)], src_idx.at[1], isems[1]).wait()
        pltpu.make_async_copy(
            dst.at[pl.ds(0, C)], dst_idx.at[1], isems[1]).wait()
        pltpu.make_async_copy(
            gate_b.at[0], gate_out.at[pl.ds(0, C)], gws[0]).wait()
        pltpu.make_async_copy(
            gate_b.at[1], gate_out.at[pl.ds(0, C)], gws[1]).wait()
        wait_scatters(0, 2)
        wait_scatters(1, 3)

        plsc.subcore_barrier()
        pltpu.sync_copy(acc_g.at[pl.ds(row0, NPT)], gs_out.at[pl.ds(row0, NPT)])
        pltpu.sync_copy(acc_m.at[pl.ds(row0, NPT)], ms_out.at[pl.ds(row0, NPT)])

    @pl.when(cid == 0)
    def _():
        run_half(tsm_lo, xd_lo, ea_lo, gate_lo, gs_lo, ms_lo)

    @pl.when(cid == 1)
    def _():
        run_half(tsm_hi, xd_hi, ea_hi, gate_hi, gs_hi, ms_hi)


def _sc_gate(src, dst, tsm_lo, tsm_hi, xd_lo, xd_hi, ea_lo, ea_hi):
    mesh = plsc.VectorSubcoreMesh(core_axis_name="c", subcore_axis_name="s")
    f = pl.kernel(
        _sc_gate_body,
        out_type=[jax.ShapeDtypeStruct((E, H), _F32)] * 2
        + [jax.ShapeDtypeStruct((NPAD, H), _F32)] * 4,
        mesh=mesh,
        scratch_types=[
            pltpu.VMEM((4, C), jnp.int32),
            pltpu.VMEM((4, C), jnp.int32),
            pltpu.VMEM((2, C, D), _F32),
            pltpu.VMEM((2, C, H), _F32),
            pltpu.VMEM((2, C, H), _F32),
            pltpu.VMEM((2, C, H), _F32),
            pltpu.VMEM((2, C, H), _F32),
            pltpu.VMEM_SHARED((NPAD, H), _F32),
            pltpu.VMEM_SHARED((NPAD, H), _F32),
            pltpu.SemaphoreType.DMA,
            pltpu.SemaphoreType.DMA,
            pltpu.SemaphoreType.DMA,
            pltpu.SemaphoreType.DMA,
            pltpu.SemaphoreType.DMA,
            pltpu.SemaphoreType.DMA,
            pltpu.SemaphoreType.DMA,
            pltpu.SemaphoreType.DMA,
            pltpu.SemaphoreType.DMA,
            pltpu.SemaphoreType.DMA,
        ],
        compiler_params=pltpu.CompilerParams(use_tc_tiling_on_sc=False),
    )
    return f(src, dst, tsm_lo, tsm_hi, xd_lo, xd_hi, ea_lo, ea_hi)


def _edge_stats_body(glo_ref, ghi_ref, w_ref, b_ref, stats_ref, h16_ref,
                     acc_ref):
    i = pl.program_id(0)

    @pl.when(i == 0)
    def _():
        acc_ref[...] = jnp.zeros_like(acc_ref)

    w = w_ref[...]
    h = (jnp.dot(glo_ref[...], w[:H, :], preferred_element_type=_F32)
         + jnp.dot(ghi_ref[...], w[H:, :], preferred_element_type=_F32)
         + b_ref[...])
    h16_ref[...] = h.astype(jnp.bfloat16)
    acc_ref[0:1, :] += jnp.sum(h, axis=0, keepdims=True)
    acc_ref[1:2, :] += jnp.sum(h * h, axis=0, keepdims=True)

    @pl.when(i == pl.num_programs(0) - 1)
    def _():
        stats_ref[...] = acc_ref[...]


def _edge_stats(gate_lo, gate_hi, W_eo, beo):
    bn = 16000
    grid = (E // bn,)
    return pl.pallas_call(
        _edge_stats_body,
        grid=grid,
        in_specs=[pl.BlockSpec((bn, H), lambda i: (i, 0)),
                  pl.BlockSpec((bn, H), lambda i: (i, 0)),
                  pl.BlockSpec((D, D), lambda i: (0, 0)),
                  pl.BlockSpec((1, D), lambda i: (0, 0))],
        out_specs=[pl.BlockSpec((8, D), lambda i: (0, 0)),
                   pl.BlockSpec((bn, D), lambda i: (i, 0))],
        out_shape=[jax.ShapeDtypeStruct((8, D), _F32),
                   jax.ShapeDtypeStruct((E, D), jnp.bfloat16)],
        scratch_shapes=[pltpu.VMEM((8, D), _F32)],
    )(gate_lo, gate_hi, W_eo, beo)


def _edge_out_body(h16_ref, stats_ref, gam_ref, bt_ref, out_ref):
    h = h16_ref[...].astype(_F32)
    mu = stats_ref[0:1, :] / E
    var = stats_ref[1:2, :] / E - mu * mu
    scale = gam_ref[...] * lax.rsqrt(var + 1e-5)
    out_ref[...] = jnp.maximum((h - mu) * scale + bt_ref[...], 0.0)


def _edge_out(h16, stats, ge, bte):
    bn = 16000
    grid = (E // bn,)
    return pl.pallas_call(
        _edge_out_body,
        grid=grid,
        in_specs=[pl.BlockSpec((bn, D), lambda i: (i, 0)),
                  pl.BlockSpec((8, D), lambda i: (0, 0)),
                  pl.BlockSpec((1, D), lambda i: (0, 0)),
                  pl.BlockSpec((1, D), lambda i: (0, 0))],
        out_specs=pl.BlockSpec((bn, D), lambda i: (i, 0)),
        out_shape=jax.ShapeDtypeStruct((E, D), _F32),
    )(h16, stats, ge, bte)


def _node_out_body(xr_ref, gslo_ref, gshi_ref, mslo_ref, mshi_ref,
                   gam_ref, bt_ref, out_ref):
    agg_lo = mslo_ref[...] / (gslo_ref[...] + 1e-6)
    agg_hi = mshi_ref[...] / (gshi_ref[...] + 1e-6)
    t = xr_ref[...] + jnp.concatenate([agg_lo, agg_hi], axis=1)
    mu = jnp.mean(t, axis=0, keepdims=True)
    var = jnp.mean((t - mu) ** 2, axis=0, keepdims=True)
    norm = gam_ref[...] * (t - mu) * lax.rsqrt(var + 1e-5) + bt_ref[...]
    out_ref[...] = jnp.maximum(norm, 0.0)


def _node_out(xr, gs_lo, gs_hi, ms_lo, ms_hi, gn, btn):
    half = pl.BlockSpec((N, H), lambda i: (0, 0))
    return pl.pallas_call(
        _node_out_body,
        grid=(1,),
        in_specs=[pl.BlockSpec((N, D), lambda i: (0, 0)),
                  half, half, half, half,
                  pl.BlockSpec((1, D), lambda i: (0, 0)),
                  pl.BlockSpec((1, D), lambda i: (0, 0))],
        out_specs=pl.BlockSpec((N, D), lambda i: (0, 0)),
        out_shape=jax.ShapeDtypeStruct((N, D), _F32),
    )(xr, gs_lo, gs_hi, ms_lo, ms_hi, gn, btn)


@jax.jit
def kernel(x, edge_index, edge_attr, W_src, b_src, W_dst, b_dst, W_edge,
           b_edge, W_msg, b_msg, W_res, b_res, W_eo, b_eo, gamma_n, beta_n,
           gamma_e, beta_e):
    r = lambda b: b.reshape(1, D)
    tsm_lo, tsm_hi, xd_lo, xd_hi, xr = _node_fwd(
        x, W_src, W_dst, W_msg, W_res, r(b_src), r(b_dst), r(b_msg), r(b_res))
    ea_lo, ea_hi = _edge_fwd(edge_attr, W_edge, r(b_edge))
    gate_lo, gate_hi, gs_lo, gs_hi, ms_lo, ms_hi = _sc_gate(
        edge_index[0], edge_index[1], tsm_lo, tsm_hi, xd_lo, xd_hi,
        ea_lo, ea_hi)
    stats, h16 = _edge_stats(gate_lo, gate_hi, W_eo, r(b_eo))
    edge_new = _edge_out(h16, stats, r(gamma_e), r(beta_e))
    x_out = _node_out(xr, gs_lo, gs_hi, ms_lo, ms_hi, r(gamma_n), r(beta_n))
    return (x_out, edge_new)
